# Initial kernel scaffold; baseline (speedup 1.0000x reference)
import jax
import jax.numpy as jnp
from jax.experimental import pallas as pl


def kernel(x, condition, params, batch):
    raise NotImplementedError("write your pallas kernel here")


# speedup vs baseline: 2.6081x; 2.6081x over previous
import jax
import jax.numpy as jnp
from jax.experimental import pallas as pl

_NODES = [30, 6, 1]


def _bn2(h, g, b):
    mu = jnp.mean(h, axis=0, keepdims=True)
    var = jnp.var(h, axis=0, keepdims=True)
    return g * (h - mu) / jnp.sqrt(var + 1e-5) + b


def _ffn2(p, x, final_linear):
    h = x @ p['W1']
    if 'b1' in p:
        h = h + p['b1']
    if 'g1' in p:
        h = _bn2(h, p['g1'], p['be1'])
    h = jax.nn.leaky_relu(h, 0.01)
    h = h @ p['W2']
    if 'b2' in p:
        h = h + p['b2']
    if not final_linear:
        if 'g2' in p:
            h = _bn2(h, p['g2'], p['be2'])
        h = jax.nn.leaky_relu(h, 0.01)
    return h


def _knn_nbr(x3, n):
    pos = x3[..., :2]
    d = jnp.sum((pos[:, :, None, :] - pos[:, None, :, :]) ** 2, -1)
    d = d + jnp.eye(n, dtype=x3.dtype) * 1e10
    _, nbr = jax.lax.top_k(-d, 5)
    return nbr


def _agg5(z2, nbr, g, n):
    z3 = z2.reshape(g, n, z2.shape[-1])
    zn = jnp.take_along_axis(z3, nbr.reshape(g, n * 5)[:, :, None], axis=1)
    zn = zn.reshape(g, n, 5, z2.shape[-1])
    return zn.sum(2).reshape(g * n, z2.shape[-1])


def _cnu2(p, x2, g, n):
    xe = _ffn2(p['emb'], x2, False)
    xa = xe.reshape(g, n, 4).sum(1)
    xg = _ffn2(p['glob'], xa, False)
    xgb = jnp.broadcast_to(xg[:, None, :], (g, n, 5)).reshape(g * n, 5)
    return x2 + _ffn2(p['out'], jnp.concatenate([xe, xgb], 1), True)


def _tsum2(p, x2, g, n):
    for cp in p['cnu']:
        x2 = x2 + _cnu2(cp, x2, g, n)
    return _ffn2(p['disc'], x2.reshape(g, n, x2.shape[-1]).sum(1), True)


def kernel(x, condition, params, batch):
    del batch
    g = condition.shape[0]
    n = _NODES[0]
    f = x.shape[-1]
    lat = [x.reshape(g, n, f).sum(1), x.reshape(g, n, f).max(1)]
    x_disc = jnp.zeros((g, 1), x.dtype)
    for l in range(2):
        nbr = _knn_nbr(x.reshape(g, n, x.shape[-1]), n)
        ep = params['emb'][l]
        cb = jnp.broadcast_to(condition[:, None, :], (g, n, 5)).reshape(g * n, 5)
        z = jnp.concatenate([x, cb], 1)
        h = _ffn2(ep['mpl0'], z + _agg5(z, nbr, g, n), False)
        z1 = jnp.concatenate([h, cb], 1)
        h = h + _ffn2(ep['mpl1'], z1 + _agg5(z1, nbr, g, n), False)
        x = _ffn2(ep['out'], h, True)
        f = x.shape[-1]
        lat.append(x.reshape(g, n, f).sum(1)); lat.append(x.reshape(g, n, f).max(1))
        x_disc = x_disc + _tsum2(params['disc'][l], x, g, n)
        pp = params['pool'][l]
        score = (x @ pp['W'] + pp['b']).reshape(g, n)
        r = _NODES[l + 1]
        _, idx = jax.lax.top_k(score, r)
        x = jnp.take_along_axis(x.reshape(g, n, f), idx[:, :, None], axis=1).reshape(g * r, f)
        n = r
    x_disc = x_disc + _tsum2(params['disc_last'], x, g, n)
    return (x_disc, jnp.hstack(lat))


# speedup vs baseline: 46.9010x; 17.9825x over previous
import functools

import jax
import jax.numpy as jnp
from jax.experimental import pallas as pl

F32 = jnp.float32


def _leaky(x):
    return jnp.where(x >= 0, x, 0.01 * x)


def _dot(a, w):
    return jnp.dot(a, w, preferred_element_type=F32)


def _bn_apply(h, s1, s2, g, be, count):
    if h.ndim == 3:
        s1, s2, g, be = s1[None], s2[None], g[None], be[None]
    mu = s1 / count
    var = s2 / count - mu * mu
    return g * (h - mu) / jnp.sqrt(var + 1e-5) + be


def _acc2(s1_ref, s2_ref, v):
    s1c = jnp.sum(v, axis=0, keepdims=True)
    s2c = jnp.sum(v * v, axis=0, keepdims=True)
    first = pl.program_id(0) == 0

    @pl.when(first)
    def _():
        s1_ref[...] = s1c
        s2_ref[...] = s2c

    @pl.when(jnp.logical_not(first))
    def _():
        s1_ref[...] = s1_ref[...] + s1c
        s2_ref[...] = s2_ref[...] + s2c


def _rowmask3(bg, npad, nreal):
    return jax.lax.broadcasted_iota(jnp.int32, (bg, npad, 1), 1) < nreal


def _agg_exact(a, z):
    z1 = z.astype(jnp.bfloat16).astype(F32)
    r = z - z1
    z2 = r.astype(jnp.bfloat16).astype(F32)
    z3 = r - z2
    dn = (((2,), (1,)), ((0,), (0,)))
    p1 = jax.lax.dot_general(a, z1, dn, preferred_element_type=F32)
    p2 = jax.lax.dot_general(a, z2, dn, preferred_element_type=F32)
    p3 = jax.lax.dot_general(a, z3, dn, preferred_element_type=F32)
    return (p1 + p2) + p3


def _k1(x3_ref, cond_ref, w1_ref, m1_ref, nbr_ref, l0s_ref, l0m_ref,
        s1_ref, s2_ref):
    bg = x3_ref.shape[0]
    x3 = x3_ref[...]
    cond = cond_ref[...]
    rm3 = _rowmask3(bg, 32, 30)
    cb = jnp.broadcast_to(cond[:, None, :], (bg, 32, 5))
    z3 = jnp.where(rm3, jnp.concatenate([x3, cb], -1), 0.0)
    px = x3[:, :, 0]
    py = x3[:, :, 1]
    dx = px[:, :, None] - px[:, None, :]
    dy = py[:, :, None] - py[:, None, :]
    d = dx * dx + dy * dy
    ii = jax.lax.broadcasted_iota(jnp.int32, (bg, 32, 32), 1)
    jj = jax.lax.broadcasted_iota(jnp.int32, (bg, 32, 32), 2)
    d = d + jnp.where(ii == jj, 1e10, 0.0)
    d = jnp.where(jj >= 30, 1e30, d)
    jf = jj.astype(F32)
    rem = d
    amat = jnp.zeros((bg, 32, 32), F32)
    nbrs = []
    for _ in range(5):
        mn = jnp.min(rem, axis=-1, keepdims=True)
        jm = jnp.min(jnp.where(rem == mn, jf, 127.0), axis=-1, keepdims=True)
        sel = (jf == jm).astype(F32)
        nbrs.append(jm)
        amat = amat + sel
        rem = jnp.where(sel > 0, 1e30, rem)
    agg = _agg_exact(amat, z3)
    nbr_ref[...] = jnp.concatenate(nbrs + [jnp.zeros((bg, 32, 3), F32)], -1)
    gin = (z3 + agg).reshape(bg * 32, 8)
    m1 = _dot(gin, w1_ref[...])
    m1 = jnp.where(rm3.reshape(bg * 32, 1), m1, 0.0)
    m1_ref[...] = m1
    l0s_ref[...] = jnp.sum(jnp.where(rm3, x3, 0.0), axis=1)
    l0m_ref[...] = jnp.max(jnp.where(rm3, x3, -1e30), axis=1)
    _acc2(s1_ref, s2_ref, m1)


def _kbn2(h_ref, s1_ref, s2_ref, g_ref, be_ref, w2_ref, o_ref, t1_ref, t2_ref,
          *, count, nreal, nmod):
    h = h_ref[...]
    a = _leaky(_bn_apply(h, s1_ref[...], s2_ref[...], g_ref[...], be_ref[...],
                         count))
    o = _dot(a, w2_ref[...])
    rows = o.shape[0]
    rm = (jax.lax.broadcasted_iota(jnp.int32, (rows, 1), 0) % nmod) < nreal
    o = jnp.where(rm, o, 0.0)
    o_ref[...] = o
    _acc2(t1_ref, t2_ref, o)


def _k3(m2_ref, t1_ref, t2_ref, g_ref, be_ref, cond_ref, nbr_ref, w1_ref,
        h0_ref, n1_ref, u1_ref, u2_ref, *, count):
    bg = m2_ref.shape[0]
    m2 = m2_ref[...]
    h0 = _leaky(_bn_apply(m2, t1_ref[...], t2_ref[...], g_ref[...],
                          be_ref[...], count))
    rm3 = _rowmask3(bg, 32, 30)
    h0_ref[...] = jnp.where(rm3, h0, 0.0).reshape(bg * 32, 10)
    cond = cond_ref[...]
    cb = jnp.broadcast_to(cond[:, None, :], (bg, 32, 5))
    z1 = jnp.where(rm3, jnp.concatenate([h0, cb], -1), 0.0)
    jj = jax.lax.broadcasted_iota(jnp.int32, (bg, 32, 32), 2)
    jf = jj.astype(F32)
    nbr = nbr_ref[...]
    amat = jnp.zeros((bg, 32, 32), F32)
    for m in range(5):
        amat = amat + (jf == nbr[:, :, m:m + 1]).astype(F32)
    agg = _agg_exact(amat, z1)
    n1 = _dot((z1 + agg).reshape(bg * 32, 15), w1_ref[...])
    n1 = jnp.where(rm3.reshape(bg * 32, 1), n1, 0.0)
    n1_ref[...] = n1
    _acc2(u1_ref, u2_ref, n1)


def _k5(n2_ref, v1_ref, v2_ref, g_ref, be_ref, h0_ref, w_ref, b_ref,
        o1_ref, w1s_ref, w2s_ref, *, count, nreal, nmod):
    n2 = n2_ref[...]
    hh = h0_ref[...] + _leaky(_bn_apply(n2, v1_ref[...], v2_ref[...],
                                        g_ref[...], be_ref[...], count))
    o1 = _dot(hh, w_ref[...]) + b_ref[...]
    rows = o1.shape[0]
    rm = (jax.lax.broadcasted_iota(jnp.int32, (rows, 1), 0) % nmod) < nreal
    o1 = jnp.where(rm, o1, 0.0)
    o1_ref[...] = o1
    _acc2(w1s_ref, w2s_ref, o1)


def _k6(o1_ref, w1s_ref, w2s_ref, g_ref, be_ref, w2_ref, b2_ref, pw_ref,
        pb_ref, x1_ref, x2_ref, ls_ref, lm_ref,
        *, count, npad, nreal, r, fout):
    bg = o1_ref.shape[0]
    o1 = o1_ref[...].reshape(bg * npad, o1_ref.shape[2])
    a = _leaky(_bn_apply(o1, w1s_ref[...], w2s_ref[...], g_ref[...],
                         be_ref[...], count))
    x1 = _dot(a, w2_ref[...]) + b2_ref[...]
    rm3 = _rowmask3(bg, npad, nreal)
    x13 = jnp.where(rm3, x1.reshape(bg, npad, fout), 0.0)
    x1_ref[...] = x13.reshape(bg * npad, fout)
    ls_ref[...] = jnp.sum(x13, axis=1)
    lm_ref[...] = jnp.max(jnp.where(rm3, x13, -1e30), axis=1)
    score = _dot(x13.reshape(bg * npad, fout), pw_ref[...]) + pb_ref[...]
    score = score.reshape(bg, npad, 1)
    score = jnp.where(rm3, score, -1e30)
    st = jnp.transpose(score, (0, 2, 1))
    lt = jnp.sum((st > score).astype(F32), axis=-1, keepdims=True)
    ii = jax.lax.broadcasted_iota(jnp.int32, (bg, npad, npad), 1)
    jj = jax.lax.broadcasted_iota(jnp.int32, (bg, npad, npad), 2)
    eq = jnp.sum(((st == score) & (jj < ii)).astype(F32), axis=-1,
                 keepdims=True)
    rank = lt + eq
    slots = []
    for s in range(r):
        selr = (rank == float(s)).astype(F32)
        slots.append(jnp.sum(selr * x13, axis=1, keepdims=True))
    if r < 8:
        slots.append(jnp.zeros((bg, 8 - r, fout), F32))
    x2_ref[...] = jnp.concatenate(slots, axis=1)


def _ktsum(x3_ref, *refs, nreal):
    wr = refs[:14]
    xd_ref = refs[14]
    bg, npad, f = x3_ref.shape
    rm3 = _rowmask3(bg, npad, nreal)
    x2 = x3_ref[...].reshape(bg * npad, f)
    wi = 0
    for _ in range(2):
        e1, e2, g1, g2, o1, o2 = (wr[wi + k][...] for k in range(6))
        wi += 6
        xe = _leaky(_dot(_leaky(_dot(x2, e1)), e2))
        xa = jnp.sum(jnp.where(rm3, xe.reshape(bg, npad, 4), 0.0), axis=1)
        xg = _leaky(_dot(_leaky(_dot(xa, g1)), g2))
        xgb = jnp.broadcast_to(xg[:, None, :], (bg, npad, 5))
        cc = jnp.concatenate([xe, xgb.reshape(bg * npad, 5)], -1)
        o = _dot(_leaky(_dot(cc, o1)), o2)
        x2 = x2 + (x2 + o)
    d1, d2 = wr[12][...], wr[13][...]
    xa2 = jnp.sum(jnp.where(rm3, x2.reshape(bg, npad, f), 0.0), axis=1)
    xd_ref[...] = _dot(_leaky(_dot(xa2, d1)), d2)


def _klast(x_ref, *refs):
    wr = refs[:14]
    xd_ref = refs[14]
    x2 = x_ref[...]
    wi = 0
    for _ in range(2):
        e1, e2, g1, g2, o1, o2 = (wr[wi + k][...] for k in range(6))
        wi += 6
        xe = _leaky(_dot(_leaky(_dot(x2, e1)), e2))
        xg = _leaky(_dot(_leaky(_dot(xe, g1)), g2))
        o = _dot(_leaky(_dot(jnp.concatenate([xe, xg], -1), o1)), o2)
        x2 = x2 + (x2 + o)
    d1, d2 = wr[12][...], wr[13][...]
    xd_ref[...] = _dot(_leaky(_dot(x2, d1)), d2)


def _k8(x3_ref, cond_ref, w1_ref, m1_ref, s1_ref, s2_ref):
    bg = x3_ref.shape[0]
    x3 = x3_ref[...]
    cond = cond_ref[...]
    rm3 = _rowmask3(bg, 8, 6)
    cb = jnp.broadcast_to(cond[:, None, :], (bg, 8, 5))
    z3 = jnp.where(rm3, jnp.concatenate([x3, cb], -1), 0.0)
    zs = jnp.sum(z3, axis=1, keepdims=True)
    gin = jnp.where(rm3, jnp.broadcast_to(zs, z3.shape), 0.0)
    m1 = _dot(gin.reshape(bg * 8, 17), w1_ref[...])
    m1 = jnp.where(rm3.reshape(bg * 8, 1), m1, 0.0)
    m1_ref[...] = m1
    _acc2(s1_ref, s2_ref, m1)


def _k10(m2_ref, t1_ref, t2_ref, g_ref, be_ref, cond_ref, w1_ref,
         h0_ref, n1_ref, u1_ref, u2_ref, *, count):
    bg = m2_ref.shape[0]
    m2 = m2_ref[...]
    h0 = _leaky(_bn_apply(m2, t1_ref[...], t2_ref[...], g_ref[...],
                          be_ref[...], count))
    rm3 = _rowmask3(bg, 8, 6)
    h0_ref[...] = jnp.where(rm3, h0, 0.0).reshape(bg * 8, 10)
    cond = cond_ref[...]
    cb = jnp.broadcast_to(cond[:, None, :], (bg, 8, 5))
    z1 = jnp.where(rm3, jnp.concatenate([h0, cb], -1), 0.0)
    zs = jnp.sum(z1, axis=1, keepdims=True)
    gin = jnp.where(rm3, jnp.broadcast_to(zs, z1.shape), 0.0)
    n1 = _dot(gin.reshape(bg * 8, 15), w1_ref[...])
    n1 = jnp.where(rm3.reshape(bg * 8, 1), n1, 0.0)
    n1_ref[...] = n1
    _acc2(u1_ref, u2_ref, n1)


def _bspec2(rows, f):
    return pl.BlockSpec((rows, f), lambda i: (i, 0))


def _bspec3(bg, n, f):
    return pl.BlockSpec((bg, n, f), lambda i: (i, 0, 0))


def _full2(a, b):
    return pl.BlockSpec((a, b), lambda i: (0, 0))


def _full3(a, b, c):
    return pl.BlockSpec((a, b, c), lambda i: (0, 0, 0))


def _stat_out(f):
    return (pl.BlockSpec((1, f), lambda i: (0, 0)),
            jax.ShapeDtypeStruct((1, f), F32))


def _pick_bg(g):
    for c in (80, 50, 40, 25, 20, 16, 10, 8, 5, 4, 2, 1):
        if g % c == 0:
            return c
    return 1


def _row(p, k):
    return p[k].reshape(1, -1)


def _tsum_weights(tp):
    ws = []
    for cp in tp['cnu']:
        ws += [cp['emb']['W1'], cp['emb']['W2'], cp['glob']['W1'],
               cp['glob']['W2'], cp['out']['W1'], cp['out']['W2']]
    ws += [tp['disc']['W1'], tp['disc']['W2']]
    return ws


def kernel(x, condition, params, batch):
    del batch
    g = condition.shape[0]
    bg0 = _pick_bg(g)
    grid0 = (g // bg0,)
    rows0 = bg0 * 32
    cnt0 = float(30 * g)

    bgl = _pick_bg(g)
    gridl = (g // bgl,)

    x3 = jnp.pad(x.reshape(g, 30, 3), ((0, 0), (0, 2), (0, 0)))
    ep0 = params['emb'][0]

    m1, nbr, l0s, l0m, s1, s2 = pl.pallas_call(
        _k1,
        grid=grid0,
        in_specs=[_bspec3(bg0, 32, 3), _bspec2(bg0, 5), _full2(8, 10)],
        out_specs=[_bspec2(rows0, 10), _bspec3(bg0, 32, 8), _bspec2(bg0, 3),
                   _bspec2(bg0, 3), _stat_out(10)[0], _stat_out(10)[0]],
        out_shape=[jax.ShapeDtypeStruct((g * 32, 10), F32),
                   jax.ShapeDtypeStruct((g, 32, 8), F32),
                   jax.ShapeDtypeStruct((g, 3), F32),
                   jax.ShapeDtypeStruct((g, 3), F32),
                   _stat_out(10)[1], _stat_out(10)[1]],
    )(x3, condition, ep0['mpl0']['W1'])

    def bn2(hbuf, s1, s2, p, gk, bek, wk, count, nreal, nmod, fin, fo):
        blk = bg0 * nmod if nmod == 32 else bgl * nmod
        return pl.pallas_call(
            functools.partial(_kbn2, count=count, nreal=nreal, nmod=nmod),
            grid=grid0 if nmod == 32 else gridl,
            in_specs=[_bspec2(blk, fin), _full2(1, fin), _full2(1, fin),
                      _full2(1, fin), _full2(1, fin), _full2(fin, fo)],
            out_specs=[_bspec2(blk, fo), _stat_out(fo)[0], _stat_out(fo)[0]],
            out_shape=[jax.ShapeDtypeStruct((hbuf.shape[0], fo), F32),
                       _stat_out(fo)[1], _stat_out(fo)[1]],
        )(hbuf, s1, s2, _row(p, gk), _row(p, bek), p[wk])

    m2, t1, t2 = bn2(m1, s1, s2, ep0['mpl0'], 'g1', 'be1', 'W2',
                     cnt0, 30, 32, 10, 10)

    h0, n1, u1, u2 = pl.pallas_call(
        functools.partial(_k3, count=cnt0),
        grid=grid0,
        in_specs=[_bspec3(bg0, 32, 10), _full2(1, 10), _full2(1, 10),
                  _full2(1, 10), _full2(1, 10), _bspec2(bg0, 5),
                  _bspec3(bg0, 32, 8), _full2(15, 10)],
        out_specs=[_bspec2(rows0, 10), _bspec2(rows0, 10),
                   _stat_out(10)[0], _stat_out(10)[0]],
        out_shape=[jax.ShapeDtypeStruct((g * 32, 10), F32),
                   jax.ShapeDtypeStruct((g * 32, 10), F32),
                   _stat_out(10)[1], _stat_out(10)[1]],
    )(m2.reshape(g, 32, 10), t1, t2, _row(ep0['mpl0'], 'g2'),
      _row(ep0['mpl0'], 'be2'), condition, nbr, ep0['mpl1']['W1'])

    n2, v1, v2 = bn2(n1, u1, u2, ep0['mpl1'], 'g1', 'be1', 'W2',
                     cnt0, 30, 32, 10, 10)

    o1, w1s, w2s = pl.pallas_call(
        functools.partial(_k5, count=cnt0, nreal=30, nmod=32),
        grid=grid0,
        in_specs=[_bspec2(rows0, 10), _full2(1, 10), _full2(1, 10),
                  _full2(1, 10), _full2(1, 10), _bspec2(rows0, 10),
                  _full2(10, 40), _full2(1, 40)],
        out_specs=[_bspec2(rows0, 40), _stat_out(40)[0], _stat_out(40)[0]],
        out_shape=[jax.ShapeDtypeStruct((g * 32, 40), F32),
                   _stat_out(40)[1], _stat_out(40)[1]],
    )(n2, v1, v2, _row(ep0['mpl1'], 'g2'), _row(ep0['mpl1'], 'be2'), h0,
      ep0['out']['W1'], _row(ep0['out'], 'b1'))

    pp0 = params['pool'][0]
    x1, x2p, l1s, l1m = pl.pallas_call(
        functools.partial(_k6, count=cnt0, npad=32, nreal=30, r=6, fout=12),
        grid=grid0,
        in_specs=[_bspec3(bg0, 32, 40), _full2(1, 40), _full2(1, 40),
                  _full2(1, 40), _full2(1, 40), _full2(40, 12),
                  _full2(1, 12), _full2(12, 1), _full2(1, 1)],
        out_specs=[_bspec2(rows0, 12), _bspec3(bg0, 8, 12), _bspec2(bg0, 12),
                   _bspec2(bg0, 12)],
        out_shape=[jax.ShapeDtypeStruct((g * 32, 12), F32),
                   jax.ShapeDtypeStruct((g, 8, 12), F32),
                   jax.ShapeDtypeStruct((g, 12), F32),
                   jax.ShapeDtypeStruct((g, 12), F32)],
    )(o1.reshape(g, 32, 40), w1s, w2s, _row(ep0['out'], 'g1'),
      _row(ep0['out'], 'be1'), ep0['out']['W2'], _row(ep0['out'], 'b2'),
      pp0['W'], pp0['b'].reshape(1, 1))

    ts0 = _tsum_weights(params['disc'][0])
    xd0 = pl.pallas_call(
        functools.partial(_ktsum, nreal=30),
        grid=grid0,
        in_specs=[_bspec3(bg0, 32, 12)] + [_full2(*w.shape) for w in ts0],
        out_specs=[pl.BlockSpec((bg0, 1), lambda i: (i, 0))],
        out_shape=[jax.ShapeDtypeStruct((g, 1), F32)],
    )(x1.reshape(g, 32, 12), *ts0)[0]

    cnt1 = float(6 * g)
    rows1 = bgl * 8
    ep1 = params['emb'][1]

    m1b, s1b, s2b = pl.pallas_call(
        _k8,
        grid=gridl,
        in_specs=[_bspec3(bgl, 8, 12), _bspec2(bgl, 5), _full2(17, 10)],
        out_specs=[_bspec2(rows1, 10), _stat_out(10)[0], _stat_out(10)[0]],
        out_shape=[jax.ShapeDtypeStruct((g * 8, 10), F32),
                   _stat_out(10)[1], _stat_out(10)[1]],
    )(x2p, condition, ep1['mpl0']['W1'])

    m2b, t1b, t2b = bn2(m1b, s1b, s2b, ep1['mpl0'], 'g1', 'be1', 'W2',
                        cnt1, 6, 8, 10, 10)

    h0b, n1b, u1b, u2b = pl.pallas_call(
        functools.partial(_k10, count=cnt1),
        grid=gridl,
        in_specs=[_bspec3(bgl, 8, 10), _full2(1, 10), _full2(1, 10),
                  _full2(1, 10), _full2(1, 10), _bspec2(bgl, 5),
                  _full2(15, 10)],
        out_specs=[_bspec2(rows1, 10), _bspec2(rows1, 10),
                   _stat_out(10)[0], _stat_out(10)[0]],
        out_shape=[jax.ShapeDtypeStruct((g * 8, 10), F32),
                   jax.ShapeDtypeStruct((g * 8, 10), F32),
                   _stat_out(10)[1], _stat_out(10)[1]],
    )(m2b.reshape(g, 8, 10), t1b, t2b, _row(ep1['mpl0'], 'g2'),
      _row(ep1['mpl0'], 'be2'), condition, ep1['mpl1']['W1'])

    n2b, v1b, v2b = bn2(n1b, u1b, u2b, ep1['mpl1'], 'g1', 'be1', 'W2',
                        cnt1, 6, 8, 10, 10)

    o1b, w1sb, w2sb = pl.pallas_call(
        functools.partial(_k5, count=cnt1, nreal=6, nmod=8),
        grid=gridl,
        in_specs=[_bspec2(rows1, 10), _full2(1, 10), _full2(1, 10),
                  _full2(1, 10), _full2(1, 10), _bspec2(rows1, 10),
                  _full2(10, 40), _full2(1, 40)],
        out_specs=[_bspec2(rows1, 40), _stat_out(40)[0], _stat_out(40)[0]],
        out_shape=[jax.ShapeDtypeStruct((g * 8, 40), F32),
                   _stat_out(40)[1], _stat_out(40)[1]],
    )(n2b, v1b, v2b, _row(ep1['mpl1'], 'g2'), _row(ep1['mpl1'], 'be2'), h0b,
      ep1['out']['W1'], _row(ep1['out'], 'b1'))

    pp1 = params['pool'][1]
    x3out, xlastp, l2s, l2m = pl.pallas_call(
        functools.partial(_k6, count=cnt1, npad=8, nreal=6, r=1, fout=18),
        grid=gridl,
        in_specs=[_bspec3(bgl, 8, 40), _full2(1, 40), _full2(1, 40),
                  _full2(1, 40), _full2(1, 40), _full2(40, 18),
                  _full2(1, 18), _full2(18, 1), _full2(1, 1)],
        out_specs=[_bspec2(rows1, 18), _bspec3(bgl, 8, 18), _bspec2(bgl, 18),
                   _bspec2(bgl, 18)],
        out_shape=[jax.ShapeDtypeStruct((g * 8, 18), F32),
                   jax.ShapeDtypeStruct((g, 8, 18), F32),
                   jax.ShapeDtypeStruct((g, 18), F32),
                   jax.ShapeDtypeStruct((g, 18), F32)],
    )(o1b.reshape(g, 8, 40), w1sb, w2sb, _row(ep1['out'], 'g1'),
      _row(ep1['out'], 'be1'), ep1['out']['W2'], _row(ep1['out'], 'b2'),
      pp1['W'], pp1['b'].reshape(1, 1))

    ts1 = _tsum_weights(params['disc'][1])
    xd1 = pl.pallas_call(
        functools.partial(_ktsum, nreal=6),
        grid=gridl,
        in_specs=[_bspec3(bgl, 8, 18)] + [_full2(*w.shape) for w in ts1],
        out_specs=[pl.BlockSpec((bgl, 1), lambda i: (i, 0))],
        out_shape=[jax.ShapeDtypeStruct((g, 1), F32)],
    )(x3out.reshape(g, 8, 18), *ts1)[0]

    xlast = xlastp[:, 0, :]
    tsl = _tsum_weights(params['disc_last'])
    bgd = _pick_bg(g)
    xdl = pl.pallas_call(
        _klast,
        grid=(g // bgd,),
        in_specs=[_bspec2(bgd, 18)] + [_full2(*w.shape) for w in tsl],
        out_specs=[pl.BlockSpec((bgd, 1), lambda i: (i, 0))],
        out_shape=[jax.ShapeDtypeStruct((g, 1), F32)],
    )(xlast, *tsl)[0]

    x_disc = (xd0 + xd1) + xdl
    lat = jnp.hstack([l0s, l0m, l1s, l1m, l2s, l2m])
    return (x_disc, lat)


# speedup vs baseline: 57.5825x; 1.2277x over previous
import functools

import jax
import jax.numpy as jnp
from jax.experimental import pallas as pl

F32 = jnp.float32


def _leaky(x):
    return jnp.where(x >= 0, x, 0.01 * x)


def _dot(a, w):
    return jnp.dot(a, w, preferred_element_type=F32)


def _bn_apply(h, s1, s2, g, be, count):
    if h.ndim == 3:
        s1, s2, g, be = s1[None], s2[None], g[None], be[None]
    mu = s1 / count
    var = s2 / count - mu * mu
    return g * (h - mu) / jnp.sqrt(var + 1e-5) + be


def _acc2(s1_ref, s2_ref, v):
    s1c = jnp.sum(v, axis=0, keepdims=True)
    s2c = jnp.sum(v * v, axis=0, keepdims=True)
    first = pl.program_id(0) == 0

    @pl.when(first)
    def _():
        s1_ref[...] = s1c
        s2_ref[...] = s2c

    @pl.when(jnp.logical_not(first))
    def _():
        s1_ref[...] = s1_ref[...] + s1c
        s2_ref[...] = s2_ref[...] + s2c


def _rowmask3(bg, npad, nreal):
    return jax.lax.broadcasted_iota(jnp.int32, (bg, npad, 1), 1) < nreal


def _agg_exact(a, z):
    z1 = z.astype(jnp.bfloat16).astype(F32)
    r = z - z1
    z2 = r.astype(jnp.bfloat16).astype(F32)
    z3 = r - z2
    dn = (((2,), (1,)), ((0,), (0,)))
    p1 = jax.lax.dot_general(a, z1, dn, preferred_element_type=F32)
    p2 = jax.lax.dot_general(a, z2, dn, preferred_element_type=F32)
    p3 = jax.lax.dot_general(a, z3, dn, preferred_element_type=F32)
    return (p1 + p2) + p3


def _k1(x3_ref, cond_ref, w1_ref, m1_ref, nbr_ref, l0s_ref, l0m_ref,
        s1_ref, s2_ref):
    bg = x3_ref.shape[0]
    x3 = x3_ref[...]
    cond = cond_ref[...]
    rm3 = _rowmask3(bg, 32, 30)
    cb = jnp.broadcast_to(cond[:, None, :], (bg, 32, 5))
    z3 = jnp.where(rm3, jnp.concatenate([x3, cb], -1), 0.0)
    px = x3[:, :, 0]
    py = x3[:, :, 1]
    dx = px[:, :, None] - px[:, None, :]
    dy = py[:, :, None] - py[:, None, :]
    d = dx * dx + dy * dy
    ii = jax.lax.broadcasted_iota(jnp.int32, (bg, 32, 32), 1)
    jj = jax.lax.broadcasted_iota(jnp.int32, (bg, 32, 32), 2)
    d = d + jnp.where(ii == jj, 1e10, 0.0)
    d = jnp.where(jj >= 30, 1e30, d)
    jf = jj.astype(F32)
    rem = d
    amat = jnp.zeros((bg, 32, 32), F32)
    nbrs = []
    for _ in range(5):
        mn = jnp.min(rem, axis=-1, keepdims=True)
        jm = jnp.min(jnp.where(rem == mn, jf, 127.0), axis=-1, keepdims=True)
        sel = (jf == jm).astype(F32)
        nbrs.append(jm)
        amat = amat + sel
        rem = jnp.where(sel > 0, 1e30, rem)
    agg = _agg_exact(amat, z3)
    nbr_ref[...] = jnp.concatenate(nbrs + [jnp.zeros((bg, 32, 3), F32)], -1)
    gin = (z3 + agg).reshape(bg * 32, 8)
    m1 = _dot(gin, w1_ref[...])
    m1 = jnp.where(rm3.reshape(bg * 32, 1), m1, 0.0)
    m1_ref[...] = m1
    l0s_ref[...] = jnp.sum(jnp.where(rm3, x3, 0.0), axis=1)
    l0m_ref[...] = jnp.max(jnp.where(rm3, x3, -1e30), axis=1)
    _acc2(s1_ref, s2_ref, m1)


def _kbn2(h_ref, s1_ref, s2_ref, g_ref, be_ref, w2_ref, o_ref, t1_ref, t2_ref,
          *, count, nreal, nmod):
    h = h_ref[...]
    a = _leaky(_bn_apply(h, s1_ref[...], s2_ref[...], g_ref[...], be_ref[...],
                         count))
    o = _dot(a, w2_ref[...])
    rows = o.shape[0]
    rm = (jax.lax.broadcasted_iota(jnp.int32, (rows, 1), 0) % nmod) < nreal
    o = jnp.where(rm, o, 0.0)
    o_ref[...] = o
    _acc2(t1_ref, t2_ref, o)


def _k3(m2_ref, t1_ref, t2_ref, g_ref, be_ref, cond_ref, nbr_ref, w1_ref,
        h0_ref, n1_ref, u1_ref, u2_ref, *, count):
    bg = m2_ref.shape[0]
    m2 = m2_ref[...]
    h0 = _leaky(_bn_apply(m2, t1_ref[...], t2_ref[...], g_ref[...],
                          be_ref[...], count))
    rm3 = _rowmask3(bg, 32, 30)
    h0_ref[...] = jnp.where(rm3, h0, 0.0).reshape(bg * 32, 10)
    cond = cond_ref[...]
    cb = jnp.broadcast_to(cond[:, None, :], (bg, 32, 5))
    z1 = jnp.where(rm3, jnp.concatenate([h0, cb], -1), 0.0)
    jj = jax.lax.broadcasted_iota(jnp.int32, (bg, 32, 32), 2)
    jf = jj.astype(F32)
    nbr = nbr_ref[...]
    amat = jnp.zeros((bg, 32, 32), F32)
    for m in range(5):
        amat = amat + (jf == nbr[:, :, m:m + 1]).astype(F32)
    agg = _agg_exact(amat, z1)
    n1 = _dot((z1 + agg).reshape(bg * 32, 15), w1_ref[...])
    n1 = jnp.where(rm3.reshape(bg * 32, 1), n1, 0.0)
    n1_ref[...] = n1
    _acc2(u1_ref, u2_ref, n1)


def _k5(n2_ref, v1_ref, v2_ref, g_ref, be_ref, h0_ref, w_ref, b_ref,
        o1_ref, w1s_ref, w2s_ref, *, count, nreal, nmod):
    n2 = n2_ref[...]
    hh = h0_ref[...] + _leaky(_bn_apply(n2, v1_ref[...], v2_ref[...],
                                        g_ref[...], be_ref[...], count))
    o1 = _dot(hh, w_ref[...]) + b_ref[...]
    rows = o1.shape[0]
    rm = (jax.lax.broadcasted_iota(jnp.int32, (rows, 1), 0) % nmod) < nreal
    o1 = jnp.where(rm, o1, 0.0)
    o1_ref[...] = o1
    _acc2(w1s_ref, w2s_ref, o1)


def _k6(o1_ref, w1s_ref, w2s_ref, g_ref, be_ref, w2_ref, b2_ref, pw_ref,
        pb_ref, x1_ref, x2_ref, ls_ref, lm_ref,
        *, count, npad, nreal, r, fout):
    bg = o1_ref.shape[0]
    o1 = o1_ref[...].reshape(bg * npad, o1_ref.shape[2])
    a = _leaky(_bn_apply(o1, w1s_ref[...], w2s_ref[...], g_ref[...],
                         be_ref[...], count))
    x1 = _dot(a, w2_ref[...]) + b2_ref[...]
    rm3 = _rowmask3(bg, npad, nreal)
    x13 = jnp.where(rm3, x1.reshape(bg, npad, fout), 0.0)
    x1_ref[...] = x13.reshape(bg * npad, fout)
    ls_ref[...] = jnp.sum(x13, axis=1)
    lm_ref[...] = jnp.max(jnp.where(rm3, x13, -1e30), axis=1)
    score = _dot(x13.reshape(bg * npad, fout), pw_ref[...]) + pb_ref[...]
    score = score.reshape(bg, npad, 1)
    score = jnp.where(rm3, score, -1e30)
    st = jnp.transpose(score, (0, 2, 1))
    lt = jnp.sum((st > score).astype(F32), axis=-1, keepdims=True)
    ii = jax.lax.broadcasted_iota(jnp.int32, (bg, npad, npad), 1)
    jj = jax.lax.broadcasted_iota(jnp.int32, (bg, npad, npad), 2)
    eq = jnp.sum(((st == score) & (jj < ii)).astype(F32), axis=-1,
                 keepdims=True)
    rank = lt + eq
    slots = []
    for s in range(r):
        selr = (rank == float(s)).astype(F32)
        slots.append(jnp.sum(selr * x13, axis=1, keepdims=True))
    if r < 8:
        slots.append(jnp.zeros((bg, 8 - r, fout), F32))
    x2_ref[...] = jnp.concatenate(slots, axis=1)


def _ktsum(x3_ref, *refs, nreal):
    wr = refs[:14]
    xd_ref = refs[14]
    bg, npad, f = x3_ref.shape
    rm3 = _rowmask3(bg, npad, nreal)
    x2 = x3_ref[...].reshape(bg * npad, f)
    wi = 0
    for _ in range(2):
        e1, e2, g1, g2, o1, o2 = (wr[wi + k][...] for k in range(6))
        wi += 6
        xe = _leaky(_dot(_leaky(_dot(x2, e1)), e2))
        xa = jnp.sum(jnp.where(rm3, xe.reshape(bg, npad, 4), 0.0), axis=1)
        xg = _leaky(_dot(_leaky(_dot(xa, g1)), g2))
        xgb = jnp.broadcast_to(xg[:, None, :], (bg, npad, 5))
        cc = jnp.concatenate([xe, xgb.reshape(bg * npad, 5)], -1)
        o = _dot(_leaky(_dot(cc, o1)), o2)
        x2 = x2 + (x2 + o)
    d1, d2 = wr[12][...], wr[13][...]
    xa2 = jnp.sum(jnp.where(rm3, x2.reshape(bg, npad, f), 0.0), axis=1)
    xd_ref[...] = _dot(_leaky(_dot(xa2, d1)), d2)


def _klast(x_ref, *refs):
    wr = refs[:14]
    xd_ref = refs[14]
    x2 = x_ref[...]
    wi = 0
    for _ in range(2):
        e1, e2, g1, g2, o1, o2 = (wr[wi + k][...] for k in range(6))
        wi += 6
        xe = _leaky(_dot(_leaky(_dot(x2, e1)), e2))
        xg = _leaky(_dot(_leaky(_dot(xe, g1)), g2))
        o = _dot(_leaky(_dot(jnp.concatenate([xe, xg], -1), o1)), o2)
        x2 = x2 + (x2 + o)
    d1, d2 = wr[12][...], wr[13][...]
    xd_ref[...] = _dot(_leaky(_dot(x2, d1)), d2)


def _k8(x3_ref, cond_ref, w1_ref, m1_ref, s1_ref, s2_ref):
    bg = x3_ref.shape[0]
    x3 = x3_ref[...]
    cond = cond_ref[...]
    rm3 = _rowmask3(bg, 8, 6)
    cb = jnp.broadcast_to(cond[:, None, :], (bg, 8, 5))
    z3 = jnp.where(rm3, jnp.concatenate([x3, cb], -1), 0.0)
    zs = jnp.sum(z3, axis=1, keepdims=True)
    gin = jnp.where(rm3, jnp.broadcast_to(zs, z3.shape), 0.0)
    m1 = _dot(gin.reshape(bg * 8, 17), w1_ref[...])
    m1 = jnp.where(rm3.reshape(bg * 8, 1), m1, 0.0)
    m1_ref[...] = m1
    _acc2(s1_ref, s2_ref, m1)


def _k10(m2_ref, t1_ref, t2_ref, g_ref, be_ref, cond_ref, w1_ref,
         h0_ref, n1_ref, u1_ref, u2_ref, *, count):
    bg = m2_ref.shape[0]
    m2 = m2_ref[...]
    h0 = _leaky(_bn_apply(m2, t1_ref[...], t2_ref[...], g_ref[...],
                          be_ref[...], count))
    rm3 = _rowmask3(bg, 8, 6)
    h0_ref[...] = jnp.where(rm3, h0, 0.0).reshape(bg * 8, 10)
    cond = cond_ref[...]
    cb = jnp.broadcast_to(cond[:, None, :], (bg, 8, 5))
    z1 = jnp.where(rm3, jnp.concatenate([h0, cb], -1), 0.0)
    zs = jnp.sum(z1, axis=1, keepdims=True)
    gin = jnp.where(rm3, jnp.broadcast_to(zs, z1.shape), 0.0)
    n1 = _dot(gin.reshape(bg * 8, 15), w1_ref[...])
    n1 = jnp.where(rm3.reshape(bg * 8, 1), n1, 0.0)
    n1_ref[...] = n1
    _acc2(u1_ref, u2_ref, n1)


def _bspec2(rows, f):
    return pl.BlockSpec((rows, f), lambda i: (i, 0))


def _bspec3(bg, n, f):
    return pl.BlockSpec((bg, n, f), lambda i: (i, 0, 0))


def _full2(a, b):
    return pl.BlockSpec((a, b), lambda i: (0, 0))


def _full3(a, b, c):
    return pl.BlockSpec((a, b, c), lambda i: (0, 0, 0))


def _stat_out(f):
    return (pl.BlockSpec((1, f), lambda i: (0, 0)),
            jax.ShapeDtypeStruct((1, f), F32))


def _pick_bg(g, cands=(80, 50, 40, 25, 20, 16, 10, 8, 5, 4, 2, 1)):
    for c in cands:
        if g % c == 0:
            return c
    return 1


def _row(p, k):
    return p[k].reshape(1, -1)


def _tsum_weights(tp):
    ws = []
    for cp in tp['cnu']:
        ws += [cp['emb']['W1'], cp['emb']['W2'], cp['glob']['W1'],
               cp['glob']['W2'], cp['out']['W1'], cp['out']['W2']]
    ws += [tp['disc']['W1'], tp['disc']['W2']]
    return ws


def kernel(x, condition, params, batch):
    del batch
    g = condition.shape[0]
    bg0 = _pick_bg(g)
    grid0 = (g // bg0,)
    rows0 = bg0 * 32
    cnt0 = float(30 * g)

    bgl = _pick_bg(g, (400, 200, 80, 40, 16, 8, 4, 2, 1))
    gridl = (g // bgl,)

    x3 = jnp.pad(x.reshape(g, 30, 3), ((0, 0), (0, 2), (0, 0)))
    ep0 = params['emb'][0]

    m1, nbr, l0s, l0m, s1, s2 = pl.pallas_call(
        _k1,
        grid=grid0,
        in_specs=[_bspec3(bg0, 32, 3), _bspec2(bg0, 5), _full2(8, 10)],
        out_specs=[_bspec2(rows0, 10), _bspec3(bg0, 32, 8), _bspec2(bg0, 3),
                   _bspec2(bg0, 3), _stat_out(10)[0], _stat_out(10)[0]],
        out_shape=[jax.ShapeDtypeStruct((g * 32, 10), F32),
                   jax.ShapeDtypeStruct((g, 32, 8), F32),
                   jax.ShapeDtypeStruct((g, 3), F32),
                   jax.ShapeDtypeStruct((g, 3), F32),
                   _stat_out(10)[1], _stat_out(10)[1]],
    )(x3, condition, ep0['mpl0']['W1'])

    def bn2(hbuf, s1, s2, p, gk, bek, wk, count, nreal, nmod, fin, fo):
        blk = bg0 * nmod if nmod == 32 else bgl * nmod
        return pl.pallas_call(
            functools.partial(_kbn2, count=count, nreal=nreal, nmod=nmod),
            grid=grid0 if nmod == 32 else gridl,
            in_specs=[_bspec2(blk, fin), _full2(1, fin), _full2(1, fin),
                      _full2(1, fin), _full2(1, fin), _full2(fin, fo)],
            out_specs=[_bspec2(blk, fo), _stat_out(fo)[0], _stat_out(fo)[0]],
            out_shape=[jax.ShapeDtypeStruct((hbuf.shape[0], fo), F32),
                       _stat_out(fo)[1], _stat_out(fo)[1]],
        )(hbuf, s1, s2, _row(p, gk), _row(p, bek), p[wk])

    m2, t1, t2 = bn2(m1, s1, s2, ep0['mpl0'], 'g1', 'be1', 'W2',
                     cnt0, 30, 32, 10, 10)

    h0, n1, u1, u2 = pl.pallas_call(
        functools.partial(_k3, count=cnt0),
        grid=grid0,
        in_specs=[_bspec3(bg0, 32, 10), _full2(1, 10), _full2(1, 10),
                  _full2(1, 10), _full2(1, 10), _bspec2(bg0, 5),
                  _bspec3(bg0, 32, 8), _full2(15, 10)],
        out_specs=[_bspec2(rows0, 10), _bspec2(rows0, 10),
                   _stat_out(10)[0], _stat_out(10)[0]],
        out_shape=[jax.ShapeDtypeStruct((g * 32, 10), F32),
                   jax.ShapeDtypeStruct((g * 32, 10), F32),
                   _stat_out(10)[1], _stat_out(10)[1]],
    )(m2.reshape(g, 32, 10), t1, t2, _row(ep0['mpl0'], 'g2'),
      _row(ep0['mpl0'], 'be2'), condition, nbr, ep0['mpl1']['W1'])

    n2, v1, v2 = bn2(n1, u1, u2, ep0['mpl1'], 'g1', 'be1', 'W2',
                     cnt0, 30, 32, 10, 10)

    o1, w1s, w2s = pl.pallas_call(
        functools.partial(_k5, count=cnt0, nreal=30, nmod=32),
        grid=grid0,
        in_specs=[_bspec2(rows0, 10), _full2(1, 10), _full2(1, 10),
                  _full2(1, 10), _full2(1, 10), _bspec2(rows0, 10),
                  _full2(10, 40), _full2(1, 40)],
        out_specs=[_bspec2(rows0, 40), _stat_out(40)[0], _stat_out(40)[0]],
        out_shape=[jax.ShapeDtypeStruct((g * 32, 40), F32),
                   _stat_out(40)[1], _stat_out(40)[1]],
    )(n2, v1, v2, _row(ep0['mpl1'], 'g2'), _row(ep0['mpl1'], 'be2'), h0,
      ep0['out']['W1'], _row(ep0['out'], 'b1'))

    pp0 = params['pool'][0]
    x1, x2p, l1s, l1m = pl.pallas_call(
        functools.partial(_k6, count=cnt0, npad=32, nreal=30, r=6, fout=12),
        grid=grid0,
        in_specs=[_bspec3(bg0, 32, 40), _full2(1, 40), _full2(1, 40),
                  _full2(1, 40), _full2(1, 40), _full2(40, 12),
                  _full2(1, 12), _full2(12, 1), _full2(1, 1)],
        out_specs=[_bspec2(rows0, 12), _bspec3(bg0, 8, 12), _bspec2(bg0, 12),
                   _bspec2(bg0, 12)],
        out_shape=[jax.ShapeDtypeStruct((g * 32, 12), F32),
                   jax.ShapeDtypeStruct((g, 8, 12), F32),
                   jax.ShapeDtypeStruct((g, 12), F32),
                   jax.ShapeDtypeStruct((g, 12), F32)],
    )(o1.reshape(g, 32, 40), w1s, w2s, _row(ep0['out'], 'g1'),
      _row(ep0['out'], 'be1'), ep0['out']['W2'], _row(ep0['out'], 'b2'),
      pp0['W'], pp0['b'].reshape(1, 1))

    ts0 = _tsum_weights(params['disc'][0])
    xd0 = pl.pallas_call(
        functools.partial(_ktsum, nreal=30),
        grid=grid0,
        in_specs=[_bspec3(bg0, 32, 12)] + [_full2(*w.shape) for w in ts0],
        out_specs=[pl.BlockSpec((bg0, 1), lambda i: (i, 0))],
        out_shape=[jax.ShapeDtypeStruct((g, 1), F32)],
    )(x1.reshape(g, 32, 12), *ts0)[0]

    cnt1 = float(6 * g)
    rows1 = bgl * 8
    ep1 = params['emb'][1]

    m1b, s1b, s2b = pl.pallas_call(
        _k8,
        grid=gridl,
        in_specs=[_bspec3(bgl, 8, 12), _bspec2(bgl, 5), _full2(17, 10)],
        out_specs=[_bspec2(rows1, 10), _stat_out(10)[0], _stat_out(10)[0]],
        out_shape=[jax.ShapeDtypeStruct((g * 8, 10), F32),
                   _stat_out(10)[1], _stat_out(10)[1]],
    )(x2p, condition, ep1['mpl0']['W1'])

    m2b, t1b, t2b = bn2(m1b, s1b, s2b, ep1['mpl0'], 'g1', 'be1', 'W2',
                        cnt1, 6, 8, 10, 10)

    h0b, n1b, u1b, u2b = pl.pallas_call(
        functools.partial(_k10, count=cnt1),
        grid=gridl,
        in_specs=[_bspec3(bgl, 8, 10), _full2(1, 10), _full2(1, 10),
                  _full2(1, 10), _full2(1, 10), _bspec2(bgl, 5),
                  _full2(15, 10)],
        out_specs=[_bspec2(rows1, 10), _bspec2(rows1, 10),
                   _stat_out(10)[0], _stat_out(10)[0]],
        out_shape=[jax.ShapeDtypeStruct((g * 8, 10), F32),
                   jax.ShapeDtypeStruct((g * 8, 10), F32),
                   _stat_out(10)[1], _stat_out(10)[1]],
    )(m2b.reshape(g, 8, 10), t1b, t2b, _row(ep1['mpl0'], 'g2'),
      _row(ep1['mpl0'], 'be2'), condition, ep1['mpl1']['W1'])

    n2b, v1b, v2b = bn2(n1b, u1b, u2b, ep1['mpl1'], 'g1', 'be1', 'W2',
                        cnt1, 6, 8, 10, 10)

    o1b, w1sb, w2sb = pl.pallas_call(
        functools.partial(_k5, count=cnt1, nreal=6, nmod=8),
        grid=gridl,
        in_specs=[_bspec2(rows1, 10), _full2(1, 10), _full2(1, 10),
                  _full2(1, 10), _full2(1, 10), _bspec2(rows1, 10),
                  _full2(10, 40), _full2(1, 40)],
        out_specs=[_bspec2(rows1, 40), _stat_out(40)[0], _stat_out(40)[0]],
        out_shape=[jax.ShapeDtypeStruct((g * 8, 40), F32),
                   _stat_out(40)[1], _stat_out(40)[1]],
    )(n2b, v1b, v2b, _row(ep1['mpl1'], 'g2'), _row(ep1['mpl1'], 'be2'), h0b,
      ep1['out']['W1'], _row(ep1['out'], 'b1'))

    pp1 = params['pool'][1]
    x3out, xlastp, l2s, l2m = pl.pallas_call(
        functools.partial(_k6, count=cnt1, npad=8, nreal=6, r=1, fout=18),
        grid=gridl,
        in_specs=[_bspec3(bgl, 8, 40), _full2(1, 40), _full2(1, 40),
                  _full2(1, 40), _full2(1, 40), _full2(40, 18),
                  _full2(1, 18), _full2(18, 1), _full2(1, 1)],
        out_specs=[_bspec2(rows1, 18), _bspec3(bgl, 8, 18), _bspec2(bgl, 18),
                   _bspec2(bgl, 18)],
        out_shape=[jax.ShapeDtypeStruct((g * 8, 18), F32),
                   jax.ShapeDtypeStruct((g, 8, 18), F32),
                   jax.ShapeDtypeStruct((g, 18), F32),
                   jax.ShapeDtypeStruct((g, 18), F32)],
    )(o1b.reshape(g, 8, 40), w1sb, w2sb, _row(ep1['out'], 'g1'),
      _row(ep1['out'], 'be1'), ep1['out']['W2'], _row(ep1['out'], 'b2'),
      pp1['W'], pp1['b'].reshape(1, 1))

    ts1 = _tsum_weights(params['disc'][1])
    xd1 = pl.pallas_call(
        functools.partial(_ktsum, nreal=6),
        grid=gridl,
        in_specs=[_bspec3(bgl, 8, 18)] + [_full2(*w.shape) for w in ts1],
        out_specs=[pl.BlockSpec((bgl, 1), lambda i: (i, 0))],
        out_shape=[jax.ShapeDtypeStruct((g, 1), F32)],
    )(x3out.reshape(g, 8, 18), *ts1)[0]

    xlast = xlastp[:, 0, :]
    tsl = _tsum_weights(params['disc_last'])
    bgd = _pick_bg(g, (2000, 1000, 500, 250, 200, 125, 100, 80, 50, 40, 25,
                       20, 16, 10, 8, 5, 4, 2, 1))
    xdl = pl.pallas_call(
        _klast,
        grid=(g // bgd,),
        in_specs=[_bspec2(bgd, 18)] + [_full2(*w.shape) for w in tsl],
        out_specs=[pl.BlockSpec((bgd, 1), lambda i: (i, 0))],
        out_shape=[jax.ShapeDtypeStruct((g, 1), F32)],
    )(xlast, *tsl)[0]

    x_disc = (xd0 + xd1) + xdl
    lat = jnp.hstack([l0s, l0m, l1s, l1m, l2s, l2m])
    return (x_disc, lat)


# speedup vs baseline: 63.2873x; 1.0991x over previous
import functools

import jax
import jax.numpy as jnp
from jax.experimental import pallas as pl

F32 = jnp.float32


def _leaky(x):
    return jnp.where(x >= 0, x, 0.01 * x)


def _dot(a, w):
    return jnp.dot(a, w, preferred_element_type=F32)


def _bn_apply(h, s1, s2, g, be, count):
    if h.ndim == 3:
        s1, s2, g, be = s1[None], s2[None], g[None], be[None]
    mu = s1 / count
    var = s2 / count - mu * mu
    return g * (h - mu) / jnp.sqrt(var + 1e-5) + be


def _acc2(s1_ref, s2_ref, v):
    s1c = jnp.sum(v, axis=0, keepdims=True)
    s2c = jnp.sum(v * v, axis=0, keepdims=True)
    first = pl.program_id(0) == 0

    @pl.when(first)
    def _():
        s1_ref[...] = s1c
        s2_ref[...] = s2c

    @pl.when(jnp.logical_not(first))
    def _():
        s1_ref[...] = s1_ref[...] + s1c
        s2_ref[...] = s2_ref[...] + s2c


def _rowmask3(bg, npad, nreal):
    return jax.lax.broadcasted_iota(jnp.int32, (bg, npad, 1), 1) < nreal


def _agg_exact(a, z):
    z1 = z.astype(jnp.bfloat16).astype(F32)
    r = z - z1
    z2 = r.astype(jnp.bfloat16).astype(F32)
    z3 = r - z2
    dn = (((2,), (1,)), ((0,), (0,)))
    p1 = jax.lax.dot_general(a, z1, dn, preferred_element_type=F32)
    p2 = jax.lax.dot_general(a, z2, dn, preferred_element_type=F32)
    p3 = jax.lax.dot_general(a, z3, dn, preferred_element_type=F32)
    return (p1 + p2) + p3


def _k1(x3_ref, cond_ref, w1_ref, m1_ref, nbr_ref, l0s_ref, l0m_ref,
        s1_ref, s2_ref):
    bg = x3_ref.shape[0]
    x3 = x3_ref[...]
    cond = cond_ref[...]
    rm3 = _rowmask3(bg, 32, 30)
    cb = jnp.broadcast_to(cond[:, None, :], (bg, 32, 5))
    z3 = jnp.where(rm3, jnp.concatenate([x3, cb], -1), 0.0)
    px = x3[:, :, 0]
    py = x3[:, :, 1]
    dx = px[:, :, None] - px[:, None, :]
    dy = py[:, :, None] - py[:, None, :]
    d = dx * dx + dy * dy
    ii = jax.lax.broadcasted_iota(jnp.int32, (bg, 32, 32), 1)
    jj = jax.lax.broadcasted_iota(jnp.int32, (bg, 32, 32), 2)
    d = d + jnp.where(ii == jj, 1e10, 0.0)
    d = jnp.where(jj >= 30, 1e30, d)
    jf = jj.astype(F32)
    rem = d
    amat = jnp.zeros((bg, 32, 32), F32)
    nbrs = []
    for _ in range(5):
        mn = jnp.min(rem, axis=-1, keepdims=True)
        jm = jnp.min(jnp.where(rem == mn, jf, 127.0), axis=-1, keepdims=True)
        sel = (jf == jm).astype(F32)
        nbrs.append(jm)
        amat = amat + sel
        rem = jnp.where(sel > 0, 1e30, rem)
    agg = _agg_exact(amat, z3)
    nbr_ref[...] = jnp.concatenate(nbrs + [jnp.zeros((bg, 32, 3), F32)], -1)
    gin = (z3 + agg).reshape(bg * 32, 8)
    m1 = _dot(gin, w1_ref[...])
    m1 = jnp.where(rm3.reshape(bg * 32, 1), m1, 0.0)
    m1_ref[...] = m1
    l0s_ref[...] = jnp.sum(jnp.where(rm3, x3, 0.0), axis=1)
    l0m_ref[...] = jnp.max(jnp.where(rm3, x3, -1e30), axis=1)
    _acc2(s1_ref, s2_ref, m1)


def _kbn2(h_ref, s1_ref, s2_ref, g_ref, be_ref, w2_ref, o_ref, t1_ref, t2_ref,
          *, count, nreal, nmod):
    h = h_ref[...]
    a = _leaky(_bn_apply(h, s1_ref[...], s2_ref[...], g_ref[...], be_ref[...],
                         count))
    o = _dot(a, w2_ref[...])
    rows = o.shape[0]
    rm = (jax.lax.broadcasted_iota(jnp.int32, (rows, 1), 0) % nmod) < nreal
    o = jnp.where(rm, o, 0.0)
    o_ref[...] = o
    _acc2(t1_ref, t2_ref, o)


def _k3(m2_ref, t1_ref, t2_ref, g_ref, be_ref, cond_ref, nbr_ref, w1_ref,
        h0_ref, n1_ref, u1_ref, u2_ref, *, count):
    bg = m2_ref.shape[0]
    m2 = m2_ref[...]
    h0 = _leaky(_bn_apply(m2, t1_ref[...], t2_ref[...], g_ref[...],
                          be_ref[...], count))
    rm3 = _rowmask3(bg, 32, 30)
    h0_ref[...] = jnp.where(rm3, h0, 0.0).reshape(bg * 32, 10)
    cond = cond_ref[...]
    cb = jnp.broadcast_to(cond[:, None, :], (bg, 32, 5))
    z1 = jnp.where(rm3, jnp.concatenate([h0, cb], -1), 0.0)
    jj = jax.lax.broadcasted_iota(jnp.int32, (bg, 32, 32), 2)
    jf = jj.astype(F32)
    nbr = nbr_ref[...]
    amat = jnp.zeros((bg, 32, 32), F32)
    for m in range(5):
        amat = amat + (jf == nbr[:, :, m:m + 1]).astype(F32)
    agg = _agg_exact(amat, z1)
    n1 = _dot((z1 + agg).reshape(bg * 32, 15), w1_ref[...])
    n1 = jnp.where(rm3.reshape(bg * 32, 1), n1, 0.0)
    n1_ref[...] = n1
    _acc2(u1_ref, u2_ref, n1)


def _k5(n2_ref, v1_ref, v2_ref, g_ref, be_ref, h0_ref, w_ref, b_ref,
        o1_ref, w1s_ref, w2s_ref, *, count, nreal, nmod):
    n2 = n2_ref[...]
    hh = h0_ref[...] + _leaky(_bn_apply(n2, v1_ref[...], v2_ref[...],
                                        g_ref[...], be_ref[...], count))
    o1 = _dot(hh, w_ref[...]) + b_ref[...]
    rows = o1.shape[0]
    rm = (jax.lax.broadcasted_iota(jnp.int32, (rows, 1), 0) % nmod) < nreal
    o1 = jnp.where(rm, o1, 0.0)
    o1_ref[...] = o1
    _acc2(w1s_ref, w2s_ref, o1)


def _k6(o1_ref, w1s_ref, w2s_ref, g_ref, be_ref, w2_ref, b2_ref, pw_ref,
        pb_ref, x1_ref, x2_ref, ls_ref, lm_ref,
        *, count, npad, nreal, r, fout):
    bg = o1_ref.shape[0]
    o1 = o1_ref[...].reshape(bg * npad, o1_ref.shape[2])
    a = _leaky(_bn_apply(o1, w1s_ref[...], w2s_ref[...], g_ref[...],
                         be_ref[...], count))
    x1 = _dot(a, w2_ref[...]) + b2_ref[...]
    rm3 = _rowmask3(bg, npad, nreal)
    x13 = jnp.where(rm3, x1.reshape(bg, npad, fout), 0.0)
    x1_ref[...] = x13.reshape(bg * npad, fout)
    ls_ref[...] = jnp.sum(x13, axis=1)
    lm_ref[...] = jnp.max(jnp.where(rm3, x13, -1e30), axis=1)
    score = _dot(x13.reshape(bg * npad, fout), pw_ref[...]) + pb_ref[...]
    score = score.reshape(bg, npad, 1)
    score = jnp.where(rm3, score, -1e30)
    st = jnp.transpose(score, (0, 2, 1))
    lt = jnp.sum((st > score).astype(F32), axis=-1, keepdims=True)
    ii = jax.lax.broadcasted_iota(jnp.int32, (bg, npad, npad), 1)
    jj = jax.lax.broadcasted_iota(jnp.int32, (bg, npad, npad), 2)
    eq = jnp.sum(((st == score) & (jj < ii)).astype(F32), axis=-1,
                 keepdims=True)
    rank = lt + eq
    slots = []
    for s in range(r):
        selr = (rank == float(s)).astype(F32)
        slots.append(jnp.sum(selr * x13, axis=1, keepdims=True))
    if r < 8:
        slots.append(jnp.zeros((bg, 8 - r, fout), F32))
    x2_ref[...] = jnp.concatenate(slots, axis=1)


def _ktsum(x3_ref, *refs, nreal):
    wr = refs[:14]
    xd_ref = refs[14]
    bg, npad, f = x3_ref.shape
    rm3 = _rowmask3(bg, npad, nreal)
    x2 = x3_ref[...].reshape(bg * npad, f)
    wi = 0
    for _ in range(2):
        e1, e2, g1, g2, o1, o2 = (wr[wi + k][...] for k in range(6))
        wi += 6
        xe = _leaky(_dot(_leaky(_dot(x2, e1)), e2))
        xa = jnp.sum(jnp.where(rm3, xe.reshape(bg, npad, 4), 0.0), axis=1)
        xg = _leaky(_dot(_leaky(_dot(xa, g1)), g2))
        xgb = jnp.broadcast_to(xg[:, None, :], (bg, npad, 5))
        cc = jnp.concatenate([xe, xgb.reshape(bg * npad, 5)], -1)
        o = _dot(_leaky(_dot(cc, o1)), o2)
        x2 = x2 + (x2 + o)
    d1, d2 = wr[12][...], wr[13][...]
    xa2 = jnp.sum(jnp.where(rm3, x2.reshape(bg, npad, f), 0.0), axis=1)
    xd_ref[...] = _dot(_leaky(_dot(xa2, d1)), d2)


def _klast(x_ref, *refs):
    wr = refs[:14]
    xd_ref = refs[14]
    x2 = x_ref[...]
    wi = 0
    for _ in range(2):
        e1, e2, g1, g2, o1, o2 = (wr[wi + k][...] for k in range(6))
        wi += 6
        xe = _leaky(_dot(_leaky(_dot(x2, e1)), e2))
        xg = _leaky(_dot(_leaky(_dot(xe, g1)), g2))
        o = _dot(_leaky(_dot(jnp.concatenate([xe, xg], -1), o1)), o2)
        x2 = x2 + (x2 + o)
    d1, d2 = wr[12][...], wr[13][...]
    xd_ref[...] = _dot(_leaky(_dot(x2, d1)), d2)


def _k8(x3_ref, cond_ref, w1_ref, m1_ref, s1_ref, s2_ref):
    bg = x3_ref.shape[0]
    x3 = x3_ref[...]
    cond = cond_ref[...]
    rm3 = _rowmask3(bg, 8, 6)
    cb = jnp.broadcast_to(cond[:, None, :], (bg, 8, 5))
    z3 = jnp.where(rm3, jnp.concatenate([x3, cb], -1), 0.0)
    zs = jnp.sum(z3, axis=1, keepdims=True)
    gin = jnp.where(rm3, jnp.broadcast_to(zs, z3.shape), 0.0)
    m1 = _dot(gin.reshape(bg * 8, 17), w1_ref[...])
    m1 = jnp.where(rm3.reshape(bg * 8, 1), m1, 0.0)
    m1_ref[...] = m1
    _acc2(s1_ref, s2_ref, m1)


def _k10(m2_ref, t1_ref, t2_ref, g_ref, be_ref, cond_ref, w1_ref,
         h0_ref, n1_ref, u1_ref, u2_ref, *, count):
    bg = m2_ref.shape[0]
    m2 = m2_ref[...]
    h0 = _leaky(_bn_apply(m2, t1_ref[...], t2_ref[...], g_ref[...],
                          be_ref[...], count))
    rm3 = _rowmask3(bg, 8, 6)
    h0_ref[...] = jnp.where(rm3, h0, 0.0).reshape(bg * 8, 10)
    cond = cond_ref[...]
    cb = jnp.broadcast_to(cond[:, None, :], (bg, 8, 5))
    z1 = jnp.where(rm3, jnp.concatenate([h0, cb], -1), 0.0)
    zs = jnp.sum(z1, axis=1, keepdims=True)
    gin = jnp.where(rm3, jnp.broadcast_to(zs, z1.shape), 0.0)
    n1 = _dot(gin.reshape(bg * 8, 15), w1_ref[...])
    n1 = jnp.where(rm3.reshape(bg * 8, 1), n1, 0.0)
    n1_ref[...] = n1
    _acc2(u1_ref, u2_ref, n1)


def _bspec2(rows, f):
    return pl.BlockSpec((rows, f), lambda i: (i, 0))


def _bspec3(bg, n, f):
    return pl.BlockSpec((bg, n, f), lambda i: (i, 0, 0))


def _full2(a, b):
    return pl.BlockSpec((a, b), lambda i: (0, 0))


def _full3(a, b, c):
    return pl.BlockSpec((a, b, c), lambda i: (0, 0, 0))


def _stat_out(f):
    return (pl.BlockSpec((1, f), lambda i: (0, 0)),
            jax.ShapeDtypeStruct((1, f), F32))


def _pick_bg(g, cands=(80, 50, 40, 25, 20, 16, 10, 8, 5, 4, 2, 1)):
    for c in cands:
        if g % c == 0:
            return c
    return 1


def _row(p, k):
    return p[k].reshape(1, -1)


def _tsum_weights(tp):
    ws = []
    for cp in tp['cnu']:
        ws += [cp['emb']['W1'], cp['emb']['W2'], cp['glob']['W1'],
               cp['glob']['W2'], cp['out']['W1'], cp['out']['W2']]
    ws += [tp['disc']['W1'], tp['disc']['W2']]
    return ws


def kernel(x, condition, params, batch):
    del batch
    g = condition.shape[0]
    bg0 = _pick_bg(g, (200, 80, 50, 40, 25, 20, 16, 10, 8, 5, 4, 2, 1))
    grid0 = (g // bg0,)
    rows0 = bg0 * 32
    cnt0 = float(30 * g)

    bgl = _pick_bg(g, (400, 200, 80, 40, 16, 8, 4, 2, 1))
    gridl = (g // bgl,)

    x3 = jnp.pad(x.reshape(g, 30, 3), ((0, 0), (0, 2), (0, 0)))
    ep0 = params['emb'][0]

    m1, nbr, l0s, l0m, s1, s2 = pl.pallas_call(
        _k1,
        grid=grid0,
        in_specs=[_bspec3(bg0, 32, 3), _bspec2(bg0, 5), _full2(8, 10)],
        out_specs=[_bspec2(rows0, 10), _bspec3(bg0, 32, 8), _bspec2(bg0, 3),
                   _bspec2(bg0, 3), _stat_out(10)[0], _stat_out(10)[0]],
        out_shape=[jax.ShapeDtypeStruct((g * 32, 10), F32),
                   jax.ShapeDtypeStruct((g, 32, 8), F32),
                   jax.ShapeDtypeStruct((g, 3), F32),
                   jax.ShapeDtypeStruct((g, 3), F32),
                   _stat_out(10)[1], _stat_out(10)[1]],
    )(x3, condition, ep0['mpl0']['W1'])

    def bn2(hbuf, s1, s2, p, gk, bek, wk, count, nreal, nmod, fin, fo):
        blk = bg0 * nmod if nmod == 32 else bgl * nmod
        return pl.pallas_call(
            functools.partial(_kbn2, count=count, nreal=nreal, nmod=nmod),
            grid=grid0 if nmod == 32 else gridl,
            in_specs=[_bspec2(blk, fin), _full2(1, fin), _full2(1, fin),
                      _full2(1, fin), _full2(1, fin), _full2(fin, fo)],
            out_specs=[_bspec2(blk, fo), _stat_out(fo)[0], _stat_out(fo)[0]],
            out_shape=[jax.ShapeDtypeStruct((hbuf.shape[0], fo), F32),
                       _stat_out(fo)[1], _stat_out(fo)[1]],
        )(hbuf, s1, s2, _row(p, gk), _row(p, bek), p[wk])

    m2, t1, t2 = bn2(m1, s1, s2, ep0['mpl0'], 'g1', 'be1', 'W2',
                     cnt0, 30, 32, 10, 10)

    h0, n1, u1, u2 = pl.pallas_call(
        functools.partial(_k3, count=cnt0),
        grid=grid0,
        in_specs=[_bspec3(bg0, 32, 10), _full2(1, 10), _full2(1, 10),
                  _full2(1, 10), _full2(1, 10), _bspec2(bg0, 5),
                  _bspec3(bg0, 32, 8), _full2(15, 10)],
        out_specs=[_bspec2(rows0, 10), _bspec2(rows0, 10),
                   _stat_out(10)[0], _stat_out(10)[0]],
        out_shape=[jax.ShapeDtypeStruct((g * 32, 10), F32),
                   jax.ShapeDtypeStruct((g * 32, 10), F32),
                   _stat_out(10)[1], _stat_out(10)[1]],
    )(m2.reshape(g, 32, 10), t1, t2, _row(ep0['mpl0'], 'g2'),
      _row(ep0['mpl0'], 'be2'), condition, nbr, ep0['mpl1']['W1'])

    n2, v1, v2 = bn2(n1, u1, u2, ep0['mpl1'], 'g1', 'be1', 'W2',
                     cnt0, 30, 32, 10, 10)

    o1, w1s, w2s = pl.pallas_call(
        functools.partial(_k5, count=cnt0, nreal=30, nmod=32),
        grid=grid0,
        in_specs=[_bspec2(rows0, 10), _full2(1, 10), _full2(1, 10),
                  _full2(1, 10), _full2(1, 10), _bspec2(rows0, 10),
                  _full2(10, 40), _full2(1, 40)],
        out_specs=[_bspec2(rows0, 40), _stat_out(40)[0], _stat_out(40)[0]],
        out_shape=[jax.ShapeDtypeStruct((g * 32, 40), F32),
                   _stat_out(40)[1], _stat_out(40)[1]],
    )(n2, v1, v2, _row(ep0['mpl1'], 'g2'), _row(ep0['mpl1'], 'be2'), h0,
      ep0['out']['W1'], _row(ep0['out'], 'b1'))

    pp0 = params['pool'][0]
    x1, x2p, l1s, l1m = pl.pallas_call(
        functools.partial(_k6, count=cnt0, npad=32, nreal=30, r=6, fout=12),
        grid=grid0,
        in_specs=[_bspec3(bg0, 32, 40), _full2(1, 40), _full2(1, 40),
                  _full2(1, 40), _full2(1, 40), _full2(40, 12),
                  _full2(1, 12), _full2(12, 1), _full2(1, 1)],
        out_specs=[_bspec2(rows0, 12), _bspec3(bg0, 8, 12), _bspec2(bg0, 12),
                   _bspec2(bg0, 12)],
        out_shape=[jax.ShapeDtypeStruct((g * 32, 12), F32),
                   jax.ShapeDtypeStruct((g, 8, 12), F32),
                   jax.ShapeDtypeStruct((g, 12), F32),
                   jax.ShapeDtypeStruct((g, 12), F32)],
    )(o1.reshape(g, 32, 40), w1s, w2s, _row(ep0['out'], 'g1'),
      _row(ep0['out'], 'be1'), ep0['out']['W2'], _row(ep0['out'], 'b2'),
      pp0['W'], pp0['b'].reshape(1, 1))

    ts0 = _tsum_weights(params['disc'][0])
    xd0 = pl.pallas_call(
        functools.partial(_ktsum, nreal=30),
        grid=grid0,
        in_specs=[_bspec3(bg0, 32, 12)] + [_full2(*w.shape) for w in ts0],
        out_specs=[pl.BlockSpec((bg0, 1), lambda i: (i, 0))],
        out_shape=[jax.ShapeDtypeStruct((g, 1), F32)],
    )(x1.reshape(g, 32, 12), *ts0)[0]

    cnt1 = float(6 * g)
    rows1 = bgl * 8
    ep1 = params['emb'][1]

    m1b, s1b, s2b = pl.pallas_call(
        _k8,
        grid=gridl,
        in_specs=[_bspec3(bgl, 8, 12), _bspec2(bgl, 5), _full2(17, 10)],
        out_specs=[_bspec2(rows1, 10), _stat_out(10)[0], _stat_out(10)[0]],
        out_shape=[jax.ShapeDtypeStruct((g * 8, 10), F32),
                   _stat_out(10)[1], _stat_out(10)[1]],
    )(x2p, condition, ep1['mpl0']['W1'])

    m2b, t1b, t2b = bn2(m1b, s1b, s2b, ep1['mpl0'], 'g1', 'be1', 'W2',
                        cnt1, 6, 8, 10, 10)

    h0b, n1b, u1b, u2b = pl.pallas_call(
        functools.partial(_k10, count=cnt1),
        grid=gridl,
        in_specs=[_bspec3(bgl, 8, 10), _full2(1, 10), _full2(1, 10),
                  _full2(1, 10), _full2(1, 10), _bspec2(bgl, 5),
                  _full2(15, 10)],
        out_specs=[_bspec2(rows1, 10), _bspec2(rows1, 10),
                   _stat_out(10)[0], _stat_out(10)[0]],
        out_shape=[jax.ShapeDtypeStruct((g * 8, 10), F32),
                   jax.ShapeDtypeStruct((g * 8, 10), F32),
                   _stat_out(10)[1], _stat_out(10)[1]],
    )(m2b.reshape(g, 8, 10), t1b, t2b, _row(ep1['mpl0'], 'g2'),
      _row(ep1['mpl0'], 'be2'), condition, ep1['mpl1']['W1'])

    n2b, v1b, v2b = bn2(n1b, u1b, u2b, ep1['mpl1'], 'g1', 'be1', 'W2',
                        cnt1, 6, 8, 10, 10)

    o1b, w1sb, w2sb = pl.pallas_call(
        functools.partial(_k5, count=cnt1, nreal=6, nmod=8),
        grid=gridl,
        in_specs=[_bspec2(rows1, 10), _full2(1, 10), _full2(1, 10),
                  _full2(1, 10), _full2(1, 10), _bspec2(rows1, 10),
                  _full2(10, 40), _full2(1, 40)],
        out_specs=[_bspec2(rows1, 40), _stat_out(40)[0], _stat_out(40)[0]],
        out_shape=[jax.ShapeDtypeStruct((g * 8, 40), F32),
                   _stat_out(40)[1], _stat_out(40)[1]],
    )(n2b, v1b, v2b, _row(ep1['mpl1'], 'g2'), _row(ep1['mpl1'], 'be2'), h0b,
      ep1['out']['W1'], _row(ep1['out'], 'b1'))

    pp1 = params['pool'][1]
    x3out, xlastp, l2s, l2m = pl.pallas_call(
        functools.partial(_k6, count=cnt1, npad=8, nreal=6, r=1, fout=18),
        grid=gridl,
        in_specs=[_bspec3(bgl, 8, 40), _full2(1, 40), _full2(1, 40),
                  _full2(1, 40), _full2(1, 40), _full2(40, 18),
                  _full2(1, 18), _full2(18, 1), _full2(1, 1)],
        out_specs=[_bspec2(rows1, 18), _bspec3(bgl, 8, 18), _bspec2(bgl, 18),
                   _bspec2(bgl, 18)],
        out_shape=[jax.ShapeDtypeStruct((g * 8, 18), F32),
                   jax.ShapeDtypeStruct((g, 8, 18), F32),
                   jax.ShapeDtypeStruct((g, 18), F32),
                   jax.ShapeDtypeStruct((g, 18), F32)],
    )(o1b.reshape(g, 8, 40), w1sb, w2sb, _row(ep1['out'], 'g1'),
      _row(ep1['out'], 'be1'), ep1['out']['W2'], _row(ep1['out'], 'b2'),
      pp1['W'], pp1['b'].reshape(1, 1))

    ts1 = _tsum_weights(params['disc'][1])
    xd1 = pl.pallas_call(
        functools.partial(_ktsum, nreal=6),
        grid=gridl,
        in_specs=[_bspec3(bgl, 8, 18)] + [_full2(*w.shape) for w in ts1],
        out_specs=[pl.BlockSpec((bgl, 1), lambda i: (i, 0))],
        out_shape=[jax.ShapeDtypeStruct((g, 1), F32)],
    )(x3out.reshape(g, 8, 18), *ts1)[0]

    xlast = xlastp[:, 0, :]
    tsl = _tsum_weights(params['disc_last'])
    bgd = _pick_bg(g, (2000, 1000, 500, 250, 200, 125, 100, 80, 50, 40, 25,
                       20, 16, 10, 8, 5, 4, 2, 1))
    xdl = pl.pallas_call(
        _klast,
        grid=(g // bgd,),
        in_specs=[_bspec2(bgd, 18)] + [_full2(*w.shape) for w in tsl],
        out_specs=[pl.BlockSpec((bgd, 1), lambda i: (i, 0))],
        out_shape=[jax.ShapeDtypeStruct((g, 1), F32)],
    )(xlast, *tsl)[0]

    x_disc = (xd0 + xd1) + xdl
    lat = jnp.hstack([l0s, l0m, l1s, l1m, l2s, l2m])
    return (x_disc, lat)


# speedup vs baseline: 65.5238x; 1.0353x over previous
import functools

import jax
import jax.numpy as jnp
from jax.experimental import pallas as pl

F32 = jnp.float32


def _leaky(x):
    return jnp.where(x >= 0, x, 0.01 * x)


def _dot(a, w):
    return jnp.dot(a, w, preferred_element_type=F32)


def _bn_apply(h, s1, s2, g, be, count):
    if h.ndim == 3:
        s1, s2, g, be = s1[None], s2[None], g[None], be[None]
    mu = s1 / count
    var = s2 / count - mu * mu
    return g * (h - mu) / jnp.sqrt(var + 1e-5) + be


def _acc2(s1_ref, s2_ref, v):
    s1c = jnp.sum(v, axis=0, keepdims=True)
    s2c = jnp.sum(v * v, axis=0, keepdims=True)
    first = pl.program_id(0) == 0

    @pl.when(first)
    def _():
        s1_ref[...] = s1c
        s2_ref[...] = s2c

    @pl.when(jnp.logical_not(first))
    def _():
        s1_ref[...] = s1_ref[...] + s1c
        s2_ref[...] = s2_ref[...] + s2c


def _rowmask3(bg, npad, nreal):
    return jax.lax.broadcasted_iota(jnp.int32, (bg, npad, 1), 1) < nreal


def _agg_exact(a, z):
    z1 = z.astype(jnp.bfloat16).astype(F32)
    r = z - z1
    z2 = r.astype(jnp.bfloat16).astype(F32)
    z3 = r - z2
    dn = (((2,), (1,)), ((0,), (0,)))
    p1 = jax.lax.dot_general(a, z1, dn, preferred_element_type=F32)
    p2 = jax.lax.dot_general(a, z2, dn, preferred_element_type=F32)
    p3 = jax.lax.dot_general(a, z3, dn, preferred_element_type=F32)
    return (p1 + p2) + p3


def _k1(x3_ref, cond_ref, w1_ref, m1_ref, nbr_ref, l0s_ref, l0m_ref,
        s1_ref, s2_ref):
    bg = x3_ref.shape[0]
    x3 = x3_ref[...]
    cond = cond_ref[...]
    rm3 = _rowmask3(bg, 32, 30)
    cb = jnp.broadcast_to(cond[:, None, :], (bg, 32, 5))
    z3 = jnp.where(rm3, jnp.concatenate([x3, cb], -1), 0.0)
    px = x3[:, :, 0]
    py = x3[:, :, 1]
    dx = px[:, :, None] - px[:, None, :]
    dy = py[:, :, None] - py[:, None, :]
    d = dx * dx + dy * dy
    ii = jax.lax.broadcasted_iota(jnp.int32, (bg, 32, 32), 1)
    jj = jax.lax.broadcasted_iota(jnp.int32, (bg, 32, 32), 2)
    d = d + jnp.where(ii == jj, 1e10, 0.0)
    d = jnp.where(jj >= 30, 1e30, d)
    jf = jj.astype(F32)
    rem = d
    amat = jnp.zeros((bg, 32, 32), F32)
    nbrs = []
    for _ in range(5):
        mn = jnp.min(rem, axis=-1, keepdims=True)
        jm = jnp.min(jnp.where(rem == mn, jf, 127.0), axis=-1, keepdims=True)
        sel = (jf == jm).astype(F32)
        nbrs.append(jm)
        amat = amat + sel
        rem = jnp.where(sel > 0, 1e30, rem)
    agg = _agg_exact(amat, z3)
    nbr_ref[...] = jnp.concatenate(nbrs + [jnp.zeros((bg, 32, 3), F32)], -1)
    gin = (z3 + agg).reshape(bg * 32, 8)
    m1 = _dot(gin, w1_ref[...])
    m1 = jnp.where(rm3.reshape(bg * 32, 1), m1, 0.0)
    m1_ref[...] = m1
    l0s_ref[...] = jnp.sum(jnp.where(rm3, x3, 0.0), axis=1)
    l0m_ref[...] = jnp.max(jnp.where(rm3, x3, -1e30), axis=1)
    _acc2(s1_ref, s2_ref, m1)


def _kbn2(h_ref, s1_ref, s2_ref, g_ref, be_ref, w2_ref, o_ref, t1_ref, t2_ref,
          *, count, nreal, nmod):
    h = h_ref[...]
    a = _leaky(_bn_apply(h, s1_ref[...], s2_ref[...], g_ref[...], be_ref[...],
                         count))
    o = _dot(a, w2_ref[...])
    rows = o.shape[0]
    rm = (jax.lax.broadcasted_iota(jnp.int32, (rows, 1), 0) % nmod) < nreal
    o = jnp.where(rm, o, 0.0)
    o_ref[...] = o
    _acc2(t1_ref, t2_ref, o)


def _k3(m2_ref, t1_ref, t2_ref, g_ref, be_ref, cond_ref, nbr_ref, w1_ref,
        h0_ref, n1_ref, u1_ref, u2_ref, *, count):
    bg = m2_ref.shape[0]
    m2 = m2_ref[...]
    h0 = _leaky(_bn_apply(m2, t1_ref[...], t2_ref[...], g_ref[...],
                          be_ref[...], count))
    rm3 = _rowmask3(bg, 32, 30)
    h0_ref[...] = jnp.where(rm3, h0, 0.0).reshape(bg * 32, 10)
    cond = cond_ref[...]
    cb = jnp.broadcast_to(cond[:, None, :], (bg, 32, 5))
    z1 = jnp.where(rm3, jnp.concatenate([h0, cb], -1), 0.0)
    jj = jax.lax.broadcasted_iota(jnp.int32, (bg, 32, 32), 2)
    jf = jj.astype(F32)
    nbr = nbr_ref[...]
    amat = jnp.zeros((bg, 32, 32), F32)
    for m in range(5):
        amat = amat + (jf == nbr[:, :, m:m + 1]).astype(F32)
    agg = _agg_exact(amat, z1)
    n1 = _dot((z1 + agg).reshape(bg * 32, 15), w1_ref[...])
    n1 = jnp.where(rm3.reshape(bg * 32, 1), n1, 0.0)
    n1_ref[...] = n1
    _acc2(u1_ref, u2_ref, n1)


def _k5(n2_ref, v1_ref, v2_ref, g_ref, be_ref, h0_ref, w_ref, b_ref,
        o1_ref, w1s_ref, w2s_ref, *, count, nreal, nmod):
    n2 = n2_ref[...]
    hh = h0_ref[...] + _leaky(_bn_apply(n2, v1_ref[...], v2_ref[...],
                                        g_ref[...], be_ref[...], count))
    o1 = _dot(hh, w_ref[...]) + b_ref[...]
    rows = o1.shape[0]
    rm = (jax.lax.broadcasted_iota(jnp.int32, (rows, 1), 0) % nmod) < nreal
    o1 = jnp.where(rm, o1, 0.0)
    o1_ref[...] = o1
    _acc2(w1s_ref, w2s_ref, o1)


def _k6(o1_ref, w1s_ref, w2s_ref, g_ref, be_ref, w2_ref, b2_ref, pw_ref,
        pb_ref, x1_ref, x2_ref, ls_ref, lm_ref,
        *, count, npad, nreal, r, fout):
    bg = o1_ref.shape[0]
    o1 = o1_ref[...].reshape(bg * npad, o1_ref.shape[2])
    a = _leaky(_bn_apply(o1, w1s_ref[...], w2s_ref[...], g_ref[...],
                         be_ref[...], count))
    x1 = _dot(a, w2_ref[...]) + b2_ref[...]
    rm3 = _rowmask3(bg, npad, nreal)
    x13 = jnp.where(rm3, x1.reshape(bg, npad, fout), 0.0)
    x1_ref[...] = x13.reshape(bg * npad, fout)
    ls_ref[...] = jnp.sum(x13, axis=1)
    lm_ref[...] = jnp.max(jnp.where(rm3, x13, -1e30), axis=1)
    score = _dot(x13.reshape(bg * npad, fout), pw_ref[...]) + pb_ref[...]
    score = score.reshape(bg, npad, 1)
    score = jnp.where(rm3, score, -1e30)
    st = jnp.transpose(score, (0, 2, 1))
    lt = jnp.sum((st > score).astype(F32), axis=-1, keepdims=True)
    ii = jax.lax.broadcasted_iota(jnp.int32, (bg, npad, npad), 1)
    jj = jax.lax.broadcasted_iota(jnp.int32, (bg, npad, npad), 2)
    eq = jnp.sum(((st == score) & (jj < ii)).astype(F32), axis=-1,
                 keepdims=True)
    rank = lt + eq
    slots = []
    for s in range(r):
        selr = (rank == float(s)).astype(F32)
        slots.append(jnp.sum(selr * x13, axis=1, keepdims=True))
    if r < 8:
        slots.append(jnp.zeros((bg, 8 - r, fout), F32))
    x2_ref[...] = jnp.concatenate(slots, axis=1)


def _ktsum(x3_ref, *refs, nreal):
    wr = refs[:14]
    xd_ref = refs[14]
    bg, npad, f = x3_ref.shape
    rm3 = _rowmask3(bg, npad, nreal)
    x2 = x3_ref[...].reshape(bg * npad, f)
    wi = 0
    for _ in range(2):
        e1, e2, g1, g2, o1, o2 = (wr[wi + k][...] for k in range(6))
        wi += 6
        xe = _leaky(_dot(_leaky(_dot(x2, e1)), e2))
        xa = jnp.sum(jnp.where(rm3, xe.reshape(bg, npad, 4), 0.0), axis=1)
        xg = _leaky(_dot(_leaky(_dot(xa, g1)), g2))
        xgb = jnp.broadcast_to(xg[:, None, :], (bg, npad, 5))
        cc = jnp.concatenate([xe, xgb.reshape(bg * npad, 5)], -1)
        o = _dot(_leaky(_dot(cc, o1)), o2)
        x2 = x2 + (x2 + o)
    d1, d2 = wr[12][...], wr[13][...]
    xa2 = jnp.sum(jnp.where(rm3, x2.reshape(bg, npad, f), 0.0), axis=1)
    xd_ref[...] = _dot(_leaky(_dot(xa2, d1)), d2)


def _klast(x_ref, *refs):
    wr = refs[:14]
    xd_ref = refs[14]
    x2 = x_ref[...]
    wi = 0
    for _ in range(2):
        e1, e2, g1, g2, o1, o2 = (wr[wi + k][...] for k in range(6))
        wi += 6
        xe = _leaky(_dot(_leaky(_dot(x2, e1)), e2))
        xg = _leaky(_dot(_leaky(_dot(xe, g1)), g2))
        o = _dot(_leaky(_dot(jnp.concatenate([xe, xg], -1), o1)), o2)
        x2 = x2 + (x2 + o)
    d1, d2 = wr[12][...], wr[13][...]
    xd_ref[...] = _dot(_leaky(_dot(x2, d1)), d2)


def _k8(x3_ref, cond_ref, w1_ref, m1_ref, s1_ref, s2_ref):
    bg = x3_ref.shape[0]
    x3 = x3_ref[...]
    cond = cond_ref[...]
    rm3 = _rowmask3(bg, 8, 6)
    cb = jnp.broadcast_to(cond[:, None, :], (bg, 8, 5))
    z3 = jnp.where(rm3, jnp.concatenate([x3, cb], -1), 0.0)
    zs = jnp.sum(z3, axis=1, keepdims=True)
    gin = jnp.where(rm3, jnp.broadcast_to(zs, z3.shape), 0.0)
    m1 = _dot(gin.reshape(bg * 8, 17), w1_ref[...])
    m1 = jnp.where(rm3.reshape(bg * 8, 1), m1, 0.0)
    m1_ref[...] = m1
    _acc2(s1_ref, s2_ref, m1)


def _k10(m2_ref, t1_ref, t2_ref, g_ref, be_ref, cond_ref, w1_ref,
         h0_ref, n1_ref, u1_ref, u2_ref, *, count):
    bg = m2_ref.shape[0]
    m2 = m2_ref[...]
    h0 = _leaky(_bn_apply(m2, t1_ref[...], t2_ref[...], g_ref[...],
                          be_ref[...], count))
    rm3 = _rowmask3(bg, 8, 6)
    h0_ref[...] = jnp.where(rm3, h0, 0.0).reshape(bg * 8, 10)
    cond = cond_ref[...]
    cb = jnp.broadcast_to(cond[:, None, :], (bg, 8, 5))
    z1 = jnp.where(rm3, jnp.concatenate([h0, cb], -1), 0.0)
    zs = jnp.sum(z1, axis=1, keepdims=True)
    gin = jnp.where(rm3, jnp.broadcast_to(zs, z1.shape), 0.0)
    n1 = _dot(gin.reshape(bg * 8, 15), w1_ref[...])
    n1 = jnp.where(rm3.reshape(bg * 8, 1), n1, 0.0)
    n1_ref[...] = n1
    _acc2(u1_ref, u2_ref, n1)


def _bspec2(rows, f):
    return pl.BlockSpec((rows, f), lambda i: (i, 0))


def _bspec3(bg, n, f):
    return pl.BlockSpec((bg, n, f), lambda i: (i, 0, 0))


def _full2(a, b):
    return pl.BlockSpec((a, b), lambda i: (0, 0))


def _full3(a, b, c):
    return pl.BlockSpec((a, b, c), lambda i: (0, 0, 0))


def _stat_out(f):
    return (pl.BlockSpec((1, f), lambda i: (0, 0)),
            jax.ShapeDtypeStruct((1, f), F32))


def _pick_bg(g, cands=(80, 50, 40, 25, 20, 16, 10, 8, 5, 4, 2, 1)):
    for c in cands:
        if g % c == 0:
            return c
    return 1


def _row(p, k):
    return p[k].reshape(1, -1)


def _tsum_weights(tp):
    ws = []
    for cp in tp['cnu']:
        ws += [cp['emb']['W1'], cp['emb']['W2'], cp['glob']['W1'],
               cp['glob']['W2'], cp['out']['W1'], cp['out']['W2']]
    ws += [tp['disc']['W1'], tp['disc']['W2']]
    return ws


def kernel(x, condition, params, batch):
    del batch
    g = condition.shape[0]
    bg0 = _pick_bg(g, (200, 80, 50, 40, 25, 20, 16, 10, 8, 5, 4, 2, 1))
    grid0 = (g // bg0,)
    rows0 = bg0 * 32
    cnt0 = float(30 * g)

    bgl = _pick_bg(g, (1000, 400, 200, 80, 40, 16, 8, 4, 2, 1))
    gridl = (g // bgl,)

    x3 = jnp.pad(x.reshape(g, 30, 3), ((0, 0), (0, 2), (0, 0)))
    ep0 = params['emb'][0]

    m1, nbr, l0s, l0m, s1, s2 = pl.pallas_call(
        _k1,
        grid=grid0,
        in_specs=[_bspec3(bg0, 32, 3), _bspec2(bg0, 5), _full2(8, 10)],
        out_specs=[_bspec2(rows0, 10), _bspec3(bg0, 32, 8), _bspec2(bg0, 3),
                   _bspec2(bg0, 3), _stat_out(10)[0], _stat_out(10)[0]],
        out_shape=[jax.ShapeDtypeStruct((g * 32, 10), F32),
                   jax.ShapeDtypeStruct((g, 32, 8), F32),
                   jax.ShapeDtypeStruct((g, 3), F32),
                   jax.ShapeDtypeStruct((g, 3), F32),
                   _stat_out(10)[1], _stat_out(10)[1]],
    )(x3, condition, ep0['mpl0']['W1'])

    def bn2(hbuf, s1, s2, p, gk, bek, wk, count, nreal, nmod, fin, fo):
        blk = bg0 * nmod if nmod == 32 else bgl * nmod
        return pl.pallas_call(
            functools.partial(_kbn2, count=count, nreal=nreal, nmod=nmod),
            grid=grid0 if nmod == 32 else gridl,
            in_specs=[_bspec2(blk, fin), _full2(1, fin), _full2(1, fin),
                      _full2(1, fin), _full2(1, fin), _full2(fin, fo)],
            out_specs=[_bspec2(blk, fo), _stat_out(fo)[0], _stat_out(fo)[0]],
            out_shape=[jax.ShapeDtypeStruct((hbuf.shape[0], fo), F32),
                       _stat_out(fo)[1], _stat_out(fo)[1]],
        )(hbuf, s1, s2, _row(p, gk), _row(p, bek), p[wk])

    m2, t1, t2 = bn2(m1, s1, s2, ep0['mpl0'], 'g1', 'be1', 'W2',
                     cnt0, 30, 32, 10, 10)

    h0, n1, u1, u2 = pl.pallas_call(
        functools.partial(_k3, count=cnt0),
        grid=grid0,
        in_specs=[_bspec3(bg0, 32, 10), _full2(1, 10), _full2(1, 10),
                  _full2(1, 10), _full2(1, 10), _bspec2(bg0, 5),
                  _bspec3(bg0, 32, 8), _full2(15, 10)],
        out_specs=[_bspec2(rows0, 10), _bspec2(rows0, 10),
                   _stat_out(10)[0], _stat_out(10)[0]],
        out_shape=[jax.ShapeDtypeStruct((g * 32, 10), F32),
                   jax.ShapeDtypeStruct((g * 32, 10), F32),
                   _stat_out(10)[1], _stat_out(10)[1]],
    )(m2.reshape(g, 32, 10), t1, t2, _row(ep0['mpl0'], 'g2'),
      _row(ep0['mpl0'], 'be2'), condition, nbr, ep0['mpl1']['W1'])

    n2, v1, v2 = bn2(n1, u1, u2, ep0['mpl1'], 'g1', 'be1', 'W2',
                     cnt0, 30, 32, 10, 10)

    o1, w1s, w2s = pl.pallas_call(
        functools.partial(_k5, count=cnt0, nreal=30, nmod=32),
        grid=grid0,
        in_specs=[_bspec2(rows0, 10), _full2(1, 10), _full2(1, 10),
                  _full2(1, 10), _full2(1, 10), _bspec2(rows0, 10),
                  _full2(10, 40), _full2(1, 40)],
        out_specs=[_bspec2(rows0, 40), _stat_out(40)[0], _stat_out(40)[0]],
        out_shape=[jax.ShapeDtypeStruct((g * 32, 40), F32),
                   _stat_out(40)[1], _stat_out(40)[1]],
    )(n2, v1, v2, _row(ep0['mpl1'], 'g2'), _row(ep0['mpl1'], 'be2'), h0,
      ep0['out']['W1'], _row(ep0['out'], 'b1'))

    pp0 = params['pool'][0]
    x1, x2p, l1s, l1m = pl.pallas_call(
        functools.partial(_k6, count=cnt0, npad=32, nreal=30, r=6, fout=12),
        grid=grid0,
        in_specs=[_bspec3(bg0, 32, 40), _full2(1, 40), _full2(1, 40),
                  _full2(1, 40), _full2(1, 40), _full2(40, 12),
                  _full2(1, 12), _full2(12, 1), _full2(1, 1)],
        out_specs=[_bspec2(rows0, 12), _bspec3(bg0, 8, 12), _bspec2(bg0, 12),
                   _bspec2(bg0, 12)],
        out_shape=[jax.ShapeDtypeStruct((g * 32, 12), F32),
                   jax.ShapeDtypeStruct((g, 8, 12), F32),
                   jax.ShapeDtypeStruct((g, 12), F32),
                   jax.ShapeDtypeStruct((g, 12), F32)],
    )(o1.reshape(g, 32, 40), w1s, w2s, _row(ep0['out'], 'g1'),
      _row(ep0['out'], 'be1'), ep0['out']['W2'], _row(ep0['out'], 'b2'),
      pp0['W'], pp0['b'].reshape(1, 1))

    ts0 = _tsum_weights(params['disc'][0])
    xd0 = pl.pallas_call(
        functools.partial(_ktsum, nreal=30),
        grid=grid0,
        in_specs=[_bspec3(bg0, 32, 12)] + [_full2(*w.shape) for w in ts0],
        out_specs=[pl.BlockSpec((bg0, 1), lambda i: (i, 0))],
        out_shape=[jax.ShapeDtypeStruct((g, 1), F32)],
    )(x1.reshape(g, 32, 12), *ts0)[0]

    cnt1 = float(6 * g)
    rows1 = bgl * 8
    ep1 = params['emb'][1]

    m1b, s1b, s2b = pl.pallas_call(
        _k8,
        grid=gridl,
        in_specs=[_bspec3(bgl, 8, 12), _bspec2(bgl, 5), _full2(17, 10)],
        out_specs=[_bspec2(rows1, 10), _stat_out(10)[0], _stat_out(10)[0]],
        out_shape=[jax.ShapeDtypeStruct((g * 8, 10), F32),
                   _stat_out(10)[1], _stat_out(10)[1]],
    )(x2p, condition, ep1['mpl0']['W1'])

    m2b, t1b, t2b = bn2(m1b, s1b, s2b, ep1['mpl0'], 'g1', 'be1', 'W2',
                        cnt1, 6, 8, 10, 10)

    h0b, n1b, u1b, u2b = pl.pallas_call(
        functools.partial(_k10, count=cnt1),
        grid=gridl,
        in_specs=[_bspec3(bgl, 8, 10), _full2(1, 10), _full2(1, 10),
                  _full2(1, 10), _full2(1, 10), _bspec2(bgl, 5),
                  _full2(15, 10)],
        out_specs=[_bspec2(rows1, 10), _bspec2(rows1, 10),
                   _stat_out(10)[0], _stat_out(10)[0]],
        out_shape=[jax.ShapeDtypeStruct((g * 8, 10), F32),
                   jax.ShapeDtypeStruct((g * 8, 10), F32),
                   _stat_out(10)[1], _stat_out(10)[1]],
    )(m2b.reshape(g, 8, 10), t1b, t2b, _row(ep1['mpl0'], 'g2'),
      _row(ep1['mpl0'], 'be2'), condition, ep1['mpl1']['W1'])

    n2b, v1b, v2b = bn2(n1b, u1b, u2b, ep1['mpl1'], 'g1', 'be1', 'W2',
                        cnt1, 6, 8, 10, 10)

    o1b, w1sb, w2sb = pl.pallas_call(
        functools.partial(_k5, count=cnt1, nreal=6, nmod=8),
        grid=gridl,
        in_specs=[_bspec2(rows1, 10), _full2(1, 10), _full2(1, 10),
                  _full2(1, 10), _full2(1, 10), _bspec2(rows1, 10),
                  _full2(10, 40), _full2(1, 40)],
        out_specs=[_bspec2(rows1, 40), _stat_out(40)[0], _stat_out(40)[0]],
        out_shape=[jax.ShapeDtypeStruct((g * 8, 40), F32),
                   _stat_out(40)[1], _stat_out(40)[1]],
    )(n2b, v1b, v2b, _row(ep1['mpl1'], 'g2'), _row(ep1['mpl1'], 'be2'), h0b,
      ep1['out']['W1'], _row(ep1['out'], 'b1'))

    pp1 = params['pool'][1]
    x3out, xlastp, l2s, l2m = pl.pallas_call(
        functools.partial(_k6, count=cnt1, npad=8, nreal=6, r=1, fout=18),
        grid=gridl,
        in_specs=[_bspec3(bgl, 8, 40), _full2(1, 40), _full2(1, 40),
                  _full2(1, 40), _full2(1, 40), _full2(40, 18),
                  _full2(1, 18), _full2(18, 1), _full2(1, 1)],
        out_specs=[_bspec2(rows1, 18), _bspec3(bgl, 8, 18), _bspec2(bgl, 18),
                   _bspec2(bgl, 18)],
        out_shape=[jax.ShapeDtypeStruct((g * 8, 18), F32),
                   jax.ShapeDtypeStruct((g, 8, 18), F32),
                   jax.ShapeDtypeStruct((g, 18), F32),
                   jax.ShapeDtypeStruct((g, 18), F32)],
    )(o1b.reshape(g, 8, 40), w1sb, w2sb, _row(ep1['out'], 'g1'),
      _row(ep1['out'], 'be1'), ep1['out']['W2'], _row(ep1['out'], 'b2'),
      pp1['W'], pp1['b'].reshape(1, 1))

    ts1 = _tsum_weights(params['disc'][1])
    xd1 = pl.pallas_call(
        functools.partial(_ktsum, nreal=6),
        grid=gridl,
        in_specs=[_bspec3(bgl, 8, 18)] + [_full2(*w.shape) for w in ts1],
        out_specs=[pl.BlockSpec((bgl, 1), lambda i: (i, 0))],
        out_shape=[jax.ShapeDtypeStruct((g, 1), F32)],
    )(x3out.reshape(g, 8, 18), *ts1)[0]

    xlast = xlastp[:, 0, :]
    tsl = _tsum_weights(params['disc_last'])
    bgd = _pick_bg(g, (2000, 1000, 500, 250, 200, 125, 100, 80, 50, 40, 25,
                       20, 16, 10, 8, 5, 4, 2, 1))
    xdl = pl.pallas_call(
        _klast,
        grid=(g // bgd,),
        in_specs=[_bspec2(bgd, 18)] + [_full2(*w.shape) for w in tsl],
        out_specs=[pl.BlockSpec((bgd, 1), lambda i: (i, 0))],
        out_shape=[jax.ShapeDtypeStruct((g, 1), F32)],
    )(xlast, *tsl)[0]

    x_disc = (xd0 + xd1) + xdl
    lat = jnp.hstack([l0s, l0m, l1s, l1m, l2s, l2m])
    return (x_disc, lat)


# speedup vs baseline: 66.1450x; 1.0095x over previous
import functools

import jax
import jax.numpy as jnp
from jax.experimental import pallas as pl

F32 = jnp.float32


def _leaky(x):
    return jnp.where(x >= 0, x, 0.01 * x)


def _dot(a, w):
    return jnp.dot(a, w, preferred_element_type=F32)


def _bn_apply(h, s1, s2, g, be, count):
    if h.ndim == 3:
        s1, s2, g, be = s1[None], s2[None], g[None], be[None]
    mu = s1 / count
    var = s2 / count - mu * mu
    return g * (h - mu) / jnp.sqrt(var + 1e-5) + be


def _acc2(s1_ref, s2_ref, v):
    s1c = jnp.sum(v, axis=0, keepdims=True)
    s2c = jnp.sum(v * v, axis=0, keepdims=True)
    first = pl.program_id(0) == 0

    @pl.when(first)
    def _():
        s1_ref[...] = s1c
        s2_ref[...] = s2c

    @pl.when(jnp.logical_not(first))
    def _():
        s1_ref[...] = s1_ref[...] + s1c
        s2_ref[...] = s2_ref[...] + s2c


def _rowmask3(bg, npad, nreal):
    return jax.lax.broadcasted_iota(jnp.int32, (bg, npad, 1), 1) < nreal


def _agg_exact(a, z):
    z1 = z.astype(jnp.bfloat16).astype(F32)
    r = z - z1
    z2 = r.astype(jnp.bfloat16).astype(F32)
    z3 = r - z2
    dn = (((2,), (1,)), ((0,), (0,)))
    p1 = jax.lax.dot_general(a, z1, dn, preferred_element_type=F32)
    p2 = jax.lax.dot_general(a, z2, dn, preferred_element_type=F32)
    p3 = jax.lax.dot_general(a, z3, dn, preferred_element_type=F32)
    return (p1 + p2) + p3


def _k1(x3_ref, cond_ref, w1_ref, m1_ref, nbr_ref, l0s_ref, l0m_ref,
        s1_ref, s2_ref):
    bg = x3_ref.shape[0]
    x3 = x3_ref[...]
    cond = cond_ref[...]
    rm3 = _rowmask3(bg, 32, 30)
    cb = jnp.broadcast_to(cond[:, None, :], (bg, 32, 5))
    z3 = jnp.where(rm3, jnp.concatenate([x3, cb], -1), 0.0)
    px = x3[:, :, 0]
    py = x3[:, :, 1]
    dx = px[:, :, None] - px[:, None, :]
    dy = py[:, :, None] - py[:, None, :]
    d = dx * dx + dy * dy
    ii = jax.lax.broadcasted_iota(jnp.int32, (bg, 32, 32), 1)
    jj = jax.lax.broadcasted_iota(jnp.int32, (bg, 32, 32), 2)
    d = d + jnp.where(ii == jj, 1e10, 0.0)
    d = jnp.where(jj >= 30, 1e30, d)
    jf = jj.astype(F32)
    rem = d
    amat = jnp.zeros((bg, 32, 32), F32)
    nbrs = []
    for _ in range(5):
        mn = jnp.min(rem, axis=-1, keepdims=True)
        jm = jnp.min(jnp.where(rem == mn, jf, 127.0), axis=-1, keepdims=True)
        sel = (jf == jm).astype(F32)
        nbrs.append(jm)
        amat = amat + sel
        rem = jnp.where(sel > 0, 1e30, rem)
    agg = _agg_exact(amat, z3)
    nbr_ref[...] = jnp.concatenate(nbrs + [jnp.zeros((bg, 32, 3), F32)], -1)
    gin = (z3 + agg).reshape(bg * 32, 8)
    m1 = _dot(gin, w1_ref[...])
    m1 = jnp.where(rm3.reshape(bg * 32, 1), m1, 0.0)
    m1_ref[...] = m1
    l0s_ref[...] = jnp.sum(jnp.where(rm3, x3, 0.0), axis=1)
    l0m_ref[...] = jnp.max(jnp.where(rm3, x3, -1e30), axis=1)
    _acc2(s1_ref, s2_ref, m1)


def _kbn2(h_ref, s1_ref, s2_ref, g_ref, be_ref, w2_ref, o_ref, t1_ref, t2_ref,
          *, count, nreal, nmod):
    h = h_ref[...]
    a = _leaky(_bn_apply(h, s1_ref[...], s2_ref[...], g_ref[...], be_ref[...],
                         count))
    o = _dot(a, w2_ref[...])
    rows = o.shape[0]
    rm = (jax.lax.broadcasted_iota(jnp.int32, (rows, 1), 0) % nmod) < nreal
    o = jnp.where(rm, o, 0.0)
    o_ref[...] = o
    _acc2(t1_ref, t2_ref, o)


def _k3(m2_ref, t1_ref, t2_ref, g_ref, be_ref, cond_ref, nbr_ref, w1_ref,
        h0_ref, n1_ref, u1_ref, u2_ref, *, count):
    bg = m2_ref.shape[0]
    m2 = m2_ref[...]
    h0 = _leaky(_bn_apply(m2, t1_ref[...], t2_ref[...], g_ref[...],
                          be_ref[...], count))
    rm3 = _rowmask3(bg, 32, 30)
    h0_ref[...] = jnp.where(rm3, h0, 0.0).reshape(bg * 32, 10)
    cond = cond_ref[...]
    cb = jnp.broadcast_to(cond[:, None, :], (bg, 32, 5))
    z1 = jnp.where(rm3, jnp.concatenate([h0, cb], -1), 0.0)
    jj = jax.lax.broadcasted_iota(jnp.int32, (bg, 32, 32), 2)
    jf = jj.astype(F32)
    nbr = nbr_ref[...]
    amat = jnp.zeros((bg, 32, 32), F32)
    for m in range(5):
        amat = amat + (jf == nbr[:, :, m:m + 1]).astype(F32)
    agg = _agg_exact(amat, z1)
    n1 = _dot((z1 + agg).reshape(bg * 32, 15), w1_ref[...])
    n1 = jnp.where(rm3.reshape(bg * 32, 1), n1, 0.0)
    n1_ref[...] = n1
    _acc2(u1_ref, u2_ref, n1)


def _k5(n2_ref, v1_ref, v2_ref, g_ref, be_ref, h0_ref, w_ref, b_ref,
        o1_ref, w1s_ref, w2s_ref, *, count, nreal, nmod):
    n2 = n2_ref[...]
    hh = h0_ref[...] + _leaky(_bn_apply(n2, v1_ref[...], v2_ref[...],
                                        g_ref[...], be_ref[...], count))
    o1 = _dot(hh, w_ref[...]) + b_ref[...]
    rows = o1.shape[0]
    rm = (jax.lax.broadcasted_iota(jnp.int32, (rows, 1), 0) % nmod) < nreal
    o1 = jnp.where(rm, o1, 0.0)
    o1_ref[...] = o1
    _acc2(w1s_ref, w2s_ref, o1)


def _k6(o1_ref, w1s_ref, w2s_ref, g_ref, be_ref, w2_ref, b2_ref, pw_ref,
        pb_ref, x1_ref, x2_ref, ls_ref, lm_ref,
        *, count, npad, nreal, r, fout):
    bg = o1_ref.shape[0]
    o1 = o1_ref[...].reshape(bg * npad, o1_ref.shape[2])
    a = _leaky(_bn_apply(o1, w1s_ref[...], w2s_ref[...], g_ref[...],
                         be_ref[...], count))
    x1 = _dot(a, w2_ref[...]) + b2_ref[...]
    rm3 = _rowmask3(bg, npad, nreal)
    x13 = jnp.where(rm3, x1.reshape(bg, npad, fout), 0.0)
    x1_ref[...] = x13.reshape(bg * npad, fout)
    ls_ref[...] = jnp.sum(x13, axis=1)
    lm_ref[...] = jnp.max(jnp.where(rm3, x13, -1e30), axis=1)
    score = _dot(x13.reshape(bg * npad, fout), pw_ref[...]) + pb_ref[...]
    score = score.reshape(bg, npad, 1)
    score = jnp.where(rm3, score, -1e30)
    st = jnp.transpose(score, (0, 2, 1))
    lt = jnp.sum((st > score).astype(F32), axis=-1, keepdims=True)
    ii = jax.lax.broadcasted_iota(jnp.int32, (bg, npad, npad), 1)
    jj = jax.lax.broadcasted_iota(jnp.int32, (bg, npad, npad), 2)
    eq = jnp.sum(((st == score) & (jj < ii)).astype(F32), axis=-1,
                 keepdims=True)
    rank = lt + eq
    slots = []
    for s in range(r):
        selr = (rank == float(s)).astype(F32)
        slots.append(jnp.sum(selr * x13, axis=1, keepdims=True))
    if r < 8:
        slots.append(jnp.zeros((bg, 8 - r, fout), F32))
    x2_ref[...] = jnp.concatenate(slots, axis=1)


def _ktsum(x3_ref, *refs, nreal):
    wr = refs[:14]
    xd_ref = refs[14]
    bg, npad, f = x3_ref.shape
    rm3 = _rowmask3(bg, npad, nreal)
    x2 = x3_ref[...].reshape(bg * npad, f)
    wi = 0
    for _ in range(2):
        e1, e2, g1, g2, o1, o2 = (wr[wi + k][...] for k in range(6))
        wi += 6
        xe = _leaky(_dot(_leaky(_dot(x2, e1)), e2))
        xa = jnp.sum(jnp.where(rm3, xe.reshape(bg, npad, 4), 0.0), axis=1)
        xg = _leaky(_dot(_leaky(_dot(xa, g1)), g2))
        xgb = jnp.broadcast_to(xg[:, None, :], (bg, npad, 5))
        cc = jnp.concatenate([xe, xgb.reshape(bg * npad, 5)], -1)
        o = _dot(_leaky(_dot(cc, o1)), o2)
        x2 = x2 + (x2 + o)
    d1, d2 = wr[12][...], wr[13][...]
    xa2 = jnp.sum(jnp.where(rm3, x2.reshape(bg, npad, f), 0.0), axis=1)
    xd_ref[...] = _dot(_leaky(_dot(xa2, d1)), d2)


def _klast(x_ref, *refs):
    wr = refs[:14]
    xd_ref = refs[14]
    x2 = x_ref[...]
    wi = 0
    for _ in range(2):
        e1, e2, g1, g2, o1, o2 = (wr[wi + k][...] for k in range(6))
        wi += 6
        xe = _leaky(_dot(_leaky(_dot(x2, e1)), e2))
        xg = _leaky(_dot(_leaky(_dot(xe, g1)), g2))
        o = _dot(_leaky(_dot(jnp.concatenate([xe, xg], -1), o1)), o2)
        x2 = x2 + (x2 + o)
    d1, d2 = wr[12][...], wr[13][...]
    xd_ref[...] = _dot(_leaky(_dot(x2, d1)), d2)


def _k8(x3_ref, cond_ref, w1_ref, m1_ref, s1_ref, s2_ref):
    bg = x3_ref.shape[0]
    x3 = x3_ref[...]
    cond = cond_ref[...]
    rm3 = _rowmask3(bg, 8, 6)
    cb = jnp.broadcast_to(cond[:, None, :], (bg, 8, 5))
    z3 = jnp.where(rm3, jnp.concatenate([x3, cb], -1), 0.0)
    zs = jnp.sum(z3, axis=1, keepdims=True)
    gin = jnp.where(rm3, jnp.broadcast_to(zs, z3.shape), 0.0)
    m1 = _dot(gin.reshape(bg * 8, 17), w1_ref[...])
    m1 = jnp.where(rm3.reshape(bg * 8, 1), m1, 0.0)
    m1_ref[...] = m1
    _acc2(s1_ref, s2_ref, m1)


def _k10(m2_ref, t1_ref, t2_ref, g_ref, be_ref, cond_ref, w1_ref,
         h0_ref, n1_ref, u1_ref, u2_ref, *, count):
    bg = m2_ref.shape[0]
    m2 = m2_ref[...]
    h0 = _leaky(_bn_apply(m2, t1_ref[...], t2_ref[...], g_ref[...],
                          be_ref[...], count))
    rm3 = _rowmask3(bg, 8, 6)
    h0_ref[...] = jnp.where(rm3, h0, 0.0).reshape(bg * 8, 10)
    cond = cond_ref[...]
    cb = jnp.broadcast_to(cond[:, None, :], (bg, 8, 5))
    z1 = jnp.where(rm3, jnp.concatenate([h0, cb], -1), 0.0)
    zs = jnp.sum(z1, axis=1, keepdims=True)
    gin = jnp.where(rm3, jnp.broadcast_to(zs, z1.shape), 0.0)
    n1 = _dot(gin.reshape(bg * 8, 15), w1_ref[...])
    n1 = jnp.where(rm3.reshape(bg * 8, 1), n1, 0.0)
    n1_ref[...] = n1
    _acc2(u1_ref, u2_ref, n1)


def _bspec2(rows, f):
    return pl.BlockSpec((rows, f), lambda i: (i, 0))


def _bspec3(bg, n, f):
    return pl.BlockSpec((bg, n, f), lambda i: (i, 0, 0))


def _full2(a, b):
    return pl.BlockSpec((a, b), lambda i: (0, 0))


def _full3(a, b, c):
    return pl.BlockSpec((a, b, c), lambda i: (0, 0, 0))


def _stat_out(f):
    return (pl.BlockSpec((1, f), lambda i: (0, 0)),
            jax.ShapeDtypeStruct((1, f), F32))


def _pick_bg(g, cands=(80, 50, 40, 25, 20, 16, 10, 8, 5, 4, 2, 1)):
    for c in cands:
        if g % c == 0:
            return c
    return 1


def _row(p, k):
    return p[k].reshape(1, -1)


def _tsum_weights(tp):
    ws = []
    for cp in tp['cnu']:
        ws += [cp['emb']['W1'], cp['emb']['W2'], cp['glob']['W1'],
               cp['glob']['W2'], cp['out']['W1'], cp['out']['W2']]
    ws += [tp['disc']['W1'], tp['disc']['W2']]
    return ws


def kernel(x, condition, params, batch):
    del batch
    g = condition.shape[0]
    bg0 = _pick_bg(g, (200, 80, 50, 40, 25, 20, 16, 10, 8, 5, 4, 2, 1))
    grid0 = (g // bg0,)
    rows0 = bg0 * 32
    cnt0 = float(30 * g)

    bgl = _pick_bg(g, (1000, 400, 200, 80, 40, 16, 8, 4, 2, 1))
    gridl = (g // bgl,)
    bgm = _pick_bg(g, (400, 200, 80, 40, 16, 8, 4, 2, 1))
    gridm = (g // bgm,)
    rowsm = bgm * 32

    x3 = jnp.pad(x.reshape(g, 30, 3), ((0, 0), (0, 2), (0, 0)))
    ep0 = params['emb'][0]

    m1, nbr, l0s, l0m, s1, s2 = pl.pallas_call(
        _k1,
        grid=grid0,
        in_specs=[_bspec3(bg0, 32, 3), _bspec2(bg0, 5), _full2(8, 10)],
        out_specs=[_bspec2(rows0, 10), _bspec3(bg0, 32, 8), _bspec2(bg0, 3),
                   _bspec2(bg0, 3), _stat_out(10)[0], _stat_out(10)[0]],
        out_shape=[jax.ShapeDtypeStruct((g * 32, 10), F32),
                   jax.ShapeDtypeStruct((g, 32, 8), F32),
                   jax.ShapeDtypeStruct((g, 3), F32),
                   jax.ShapeDtypeStruct((g, 3), F32),
                   _stat_out(10)[1], _stat_out(10)[1]],
    )(x3, condition, ep0['mpl0']['W1'])

    def bn2(hbuf, s1, s2, p, gk, bek, wk, count, nreal, nmod, fin, fo):
        blk = bgm * nmod if nmod == 32 else bgl * nmod
        return pl.pallas_call(
            functools.partial(_kbn2, count=count, nreal=nreal, nmod=nmod),
            grid=gridm if nmod == 32 else gridl,
            in_specs=[_bspec2(blk, fin), _full2(1, fin), _full2(1, fin),
                      _full2(1, fin), _full2(1, fin), _full2(fin, fo)],
            out_specs=[_bspec2(blk, fo), _stat_out(fo)[0], _stat_out(fo)[0]],
            out_shape=[jax.ShapeDtypeStruct((hbuf.shape[0], fo), F32),
                       _stat_out(fo)[1], _stat_out(fo)[1]],
        )(hbuf, s1, s2, _row(p, gk), _row(p, bek), p[wk])

    m2, t1, t2 = bn2(m1, s1, s2, ep0['mpl0'], 'g1', 'be1', 'W2',
                     cnt0, 30, 32, 10, 10)

    h0, n1, u1, u2 = pl.pallas_call(
        functools.partial(_k3, count=cnt0),
        grid=grid0,
        in_specs=[_bspec3(bg0, 32, 10), _full2(1, 10), _full2(1, 10),
                  _full2(1, 10), _full2(1, 10), _bspec2(bg0, 5),
                  _bspec3(bg0, 32, 8), _full2(15, 10)],
        out_specs=[_bspec2(rows0, 10), _bspec2(rows0, 10),
                   _stat_out(10)[0], _stat_out(10)[0]],
        out_shape=[jax.ShapeDtypeStruct((g * 32, 10), F32),
                   jax.ShapeDtypeStruct((g * 32, 10), F32),
                   _stat_out(10)[1], _stat_out(10)[1]],
    )(m2.reshape(g, 32, 10), t1, t2, _row(ep0['mpl0'], 'g2'),
      _row(ep0['mpl0'], 'be2'), condition, nbr, ep0['mpl1']['W1'])

    n2, v1, v2 = bn2(n1, u1, u2, ep0['mpl1'], 'g1', 'be1', 'W2',
                     cnt0, 30, 32, 10, 10)

    o1, w1s, w2s = pl.pallas_call(
        functools.partial(_k5, count=cnt0, nreal=30, nmod=32),
        grid=gridm,
        in_specs=[_bspec2(rowsm, 10), _full2(1, 10), _full2(1, 10),
                  _full2(1, 10), _full2(1, 10), _bspec2(rowsm, 10),
                  _full2(10, 40), _full2(1, 40)],
        out_specs=[_bspec2(rowsm, 40), _stat_out(40)[0], _stat_out(40)[0]],
        out_shape=[jax.ShapeDtypeStruct((g * 32, 40), F32),
                   _stat_out(40)[1], _stat_out(40)[1]],
    )(n2, v1, v2, _row(ep0['mpl1'], 'g2'), _row(ep0['mpl1'], 'be2'), h0,
      ep0['out']['W1'], _row(ep0['out'], 'b1'))

    pp0 = params['pool'][0]
    x1, x2p, l1s, l1m = pl.pallas_call(
        functools.partial(_k6, count=cnt0, npad=32, nreal=30, r=6, fout=12),
        grid=grid0,
        in_specs=[_bspec3(bg0, 32, 40), _full2(1, 40), _full2(1, 40),
                  _full2(1, 40), _full2(1, 40), _full2(40, 12),
                  _full2(1, 12), _full2(12, 1), _full2(1, 1)],
        out_specs=[_bspec2(rows0, 12), _bspec3(bg0, 8, 12), _bspec2(bg0, 12),
                   _bspec2(bg0, 12)],
        out_shape=[jax.ShapeDtypeStruct((g * 32, 12), F32),
                   jax.ShapeDtypeStruct((g, 8, 12), F32),
                   jax.ShapeDtypeStruct((g, 12), F32),
                   jax.ShapeDtypeStruct((g, 12), F32)],
    )(o1.reshape(g, 32, 40), w1s, w2s, _row(ep0['out'], 'g1'),
      _row(ep0['out'], 'be1'), ep0['out']['W2'], _row(ep0['out'], 'b2'),
      pp0['W'], pp0['b'].reshape(1, 1))

    ts0 = _tsum_weights(params['disc'][0])
    xd0 = pl.pallas_call(
        functools.partial(_ktsum, nreal=30),
        grid=grid0,
        in_specs=[_bspec3(bg0, 32, 12)] + [_full2(*w.shape) for w in ts0],
        out_specs=[pl.BlockSpec((bg0, 1), lambda i: (i, 0))],
        out_shape=[jax.ShapeDtypeStruct((g, 1), F32)],
    )(x1.reshape(g, 32, 12), *ts0)[0]

    cnt1 = float(6 * g)
    rows1 = bgl * 8
    ep1 = params['emb'][1]

    m1b, s1b, s2b = pl.pallas_call(
        _k8,
        grid=gridl,
        in_specs=[_bspec3(bgl, 8, 12), _bspec2(bgl, 5), _full2(17, 10)],
        out_specs=[_bspec2(rows1, 10), _stat_out(10)[0], _stat_out(10)[0]],
        out_shape=[jax.ShapeDtypeStruct((g * 8, 10), F32),
                   _stat_out(10)[1], _stat_out(10)[1]],
    )(x2p, condition, ep1['mpl0']['W1'])

    m2b, t1b, t2b = bn2(m1b, s1b, s2b, ep1['mpl0'], 'g1', 'be1', 'W2',
                        cnt1, 6, 8, 10, 10)

    h0b, n1b, u1b, u2b = pl.pallas_call(
        functools.partial(_k10, count=cnt1),
        grid=gridl,
        in_specs=[_bspec3(bgl, 8, 10), _full2(1, 10), _full2(1, 10),
                  _full2(1, 10), _full2(1, 10), _bspec2(bgl, 5),
                  _full2(15, 10)],
        out_specs=[_bspec2(rows1, 10), _bspec2(rows1, 10),
                   _stat_out(10)[0], _stat_out(10)[0]],
        out_shape=[jax.ShapeDtypeStruct((g * 8, 10), F32),
                   jax.ShapeDtypeStruct((g * 8, 10), F32),
                   _stat_out(10)[1], _stat_out(10)[1]],
    )(m2b.reshape(g, 8, 10), t1b, t2b, _row(ep1['mpl0'], 'g2'),
      _row(ep1['mpl0'], 'be2'), condition, ep1['mpl1']['W1'])

    n2b, v1b, v2b = bn2(n1b, u1b, u2b, ep1['mpl1'], 'g1', 'be1', 'W2',
                        cnt1, 6, 8, 10, 10)

    o1b, w1sb, w2sb = pl.pallas_call(
        functools.partial(_k5, count=cnt1, nreal=6, nmod=8),
        grid=gridl,
        in_specs=[_bspec2(rows1, 10), _full2(1, 10), _full2(1, 10),
                  _full2(1, 10), _full2(1, 10), _bspec2(rows1, 10),
                  _full2(10, 40), _full2(1, 40)],
        out_specs=[_bspec2(rows1, 40), _stat_out(40)[0], _stat_out(40)[0]],
        out_shape=[jax.ShapeDtypeStruct((g * 8, 40), F32),
                   _stat_out(40)[1], _stat_out(40)[1]],
    )(n2b, v1b, v2b, _row(ep1['mpl1'], 'g2'), _row(ep1['mpl1'], 'be2'), h0b,
      ep1['out']['W1'], _row(ep1['out'], 'b1'))

    pp1 = params['pool'][1]
    x3out, xlastp, l2s, l2m = pl.pallas_call(
        functools.partial(_k6, count=cnt1, npad=8, nreal=6, r=1, fout=18),
        grid=gridl,
        in_specs=[_bspec3(bgl, 8, 40), _full2(1, 40), _full2(1, 40),
                  _full2(1, 40), _full2(1, 40), _full2(40, 18),
                  _full2(1, 18), _full2(18, 1), _full2(1, 1)],
        out_specs=[_bspec2(rows1, 18), _bspec3(bgl, 8, 18), _bspec2(bgl, 18),
                   _bspec2(bgl, 18)],
        out_shape=[jax.ShapeDtypeStruct((g * 8, 18), F32),
                   jax.ShapeDtypeStruct((g, 8, 18), F32),
                   jax.ShapeDtypeStruct((g, 18), F32),
                   jax.ShapeDtypeStruct((g, 18), F32)],
    )(o1b.reshape(g, 8, 40), w1sb, w2sb, _row(ep1['out'], 'g1'),
      _row(ep1['out'], 'be1'), ep1['out']['W2'], _row(ep1['out'], 'b2'),
      pp1['W'], pp1['b'].reshape(1, 1))

    ts1 = _tsum_weights(params['disc'][1])
    xd1 = pl.pallas_call(
        functools.partial(_ktsum, nreal=6),
        grid=gridl,
        in_specs=[_bspec3(bgl, 8, 18)] + [_full2(*w.shape) for w in ts1],
        out_specs=[pl.BlockSpec((bgl, 1), lambda i: (i, 0))],
        out_shape=[jax.ShapeDtypeStruct((g, 1), F32)],
    )(x3out.reshape(g, 8, 18), *ts1)[0]

    xlast = xlastp[:, 0, :]
    tsl = _tsum_weights(params['disc_last'])
    bgd = _pick_bg(g, (2000, 1000, 500, 250, 200, 125, 100, 80, 50, 40, 25,
                       20, 16, 10, 8, 5, 4, 2, 1))
    xdl = pl.pallas_call(
        _klast,
        grid=(g // bgd,),
        in_specs=[_bspec2(bgd, 18)] + [_full2(*w.shape) for w in tsl],
        out_specs=[pl.BlockSpec((bgd, 1), lambda i: (i, 0))],
        out_shape=[jax.ShapeDtypeStruct((g, 1), F32)],
    )(xlast, *tsl)[0]

    x_disc = (xd0 + xd1) + xdl
    lat = jnp.hstack([l0s, l0m, l1s, l1m, l2s, l2m])
    return (x_disc, lat)
